# baseline (speedup 1.0000x reference)
import functools

import jax
import jax.numpy as jnp
from jax import lax
from jax.experimental import pallas as pl
from jax.experimental.pallas import tpu as pltpu
from jax.experimental.pallas import tpu_sc as plsc

N = 10000
DIN = 128
DH = 64
E = 320000

NC = 2
NS = 16
NW = NC * NS
L = 16

NPAD = 10240
BLK = 1024
NBLK = NPAD // BLK
CW = 128
K0 = 80
K1 = 80
NCHUNKS = NS * (K0 + K1)
EPAD = NCHUNKS * CW
ROWS_PER_TILE = NPAD // NS

EW = EPAD // NW


def _deg_body(dst_hbm, out_hbm, idx_v, deg_v):
    c = lax.axis_index("c")
    s = lax.axis_index("s")
    wid = s * NC + c
    zero16 = jnp.zeros((L,), jnp.float32)

    def zb(i, carry):
        deg_v[pl.ds(i * L, L)] = zero16
        return carry

    lax.fori_loop(0, NPAD // L, zb, 0)
    pltpu.sync_copy(dst_hbm.at[pl.ds(wid * EW, EW)], idx_v)
    ones16 = jnp.ones((L,), jnp.float32)

    def body(i, carry):
        idx16 = idx_v[pl.ds(i * L, L)]
        plsc.addupdate_scatter(deg_v, [idx16], ones16)
        return carry

    lax.fori_loop(0, EW // L, body, 0)
    pltpu.sync_copy(deg_v, out_hbm.at[wid])


def _deg_counts(dst_flat):
    mesh = plsc.VectorSubcoreMesh(
        core_axis_name="c", subcore_axis_name="s", num_cores=NC, num_subcores=NS)
    f = pl.kernel(
        _deg_body,
        out_type=jax.ShapeDtypeStruct((NW, NPAD), jnp.float32),
        mesh=mesh,
        scratch_types=[
            pltpu.VMEM((EW,), jnp.int32),
            pltpu.VMEM((NPAD,), jnp.float32),
        ],
        compiler_params=pltpu.CompilerParams(
            needs_layout_passes=False, use_tc_tiling_on_sc=False),
    )
    return f(dst_flat)


def _row_scale(dinv_row, a):
    n = dinv_row.shape[1]
    outs = []
    rr = lax.broadcasted_iota(jnp.int32, (128, 128), 0)
    qq = lax.broadcasted_iota(jnp.int32, (128, 128), 1)
    eye = rr == qq
    for r in range(n // 128):
        dsub = dinv_row[:, r * 128:(r + 1) * 128]
        dg = jnp.where(eye, jnp.broadcast_to(dsub, (128, 128)), 0.0)
        outs.append(jnp.dot(dg, a[r * 128:(r + 1) * 128],
                            preferred_element_type=jnp.float32))
    return jnp.concatenate(outs, axis=0)


def _k2_body(x_ref, w1_ref, degp_ref, ht_ref, dinv_ref):
    deg = jnp.sum(degp_ref[...], axis=0, keepdims=True) + 1.0
    dinv = lax.rsqrt(deg)
    dinv_ref[...] = dinv.reshape(1, 1, BLK)
    h = jnp.dot(x_ref[...], w1_ref[...], preferred_element_type=jnp.float32)
    ht_ref[...] = _row_scale(dinv, h)


def _scale_stage(x_pad, W1, deg_part):
    return pl.pallas_call(
        _k2_body,
        grid=(NBLK,),
        in_specs=[
            pl.BlockSpec((BLK, DIN), lambda i: (i, 0)),
            pl.BlockSpec((DIN, DH), lambda i: (0, 0)),
            pl.BlockSpec((NW, BLK), lambda i: (0, i)),
        ],
        out_specs=[
            pl.BlockSpec((BLK, DH), lambda i: (i, 0)),
            pl.BlockSpec((1, 1, BLK), lambda i: (i, 0, 0)),
        ],
        out_shape=[
            jax.ShapeDtypeStruct((NPAD, DH), jnp.float32),
            jax.ShapeDtypeStruct((NBLK, 1, BLK), jnp.float32),
        ],
        compiler_params=pltpu.CompilerParams(
            dimension_semantics=("arbitrary",)),
    )(x_pad, W1, deg_part)


def _edge_body(src_hbm, dst_hbm, ht_hbm, dinv_hbm, sagg_hbm, spart_hbm,
               src_v, dst_v, rows_v, dinv_v, s_v,
               g0, g1, g2, g3, s0, s1, s2, s3, agg_sh):
    c = lax.axis_index("c")
    s = lax.axis_index("s")
    wid = s * NC + c
    gsem = [g0, g1, g2, g3]
    ssem = [s0, s1, s2, s3]
    zero16 = jnp.zeros((L,), jnp.float32)

    base = jnp.where(c == 0, s * K0, NS * K0 + s * K1)
    k = jnp.where(c == 0, K0, K1)

    def zrows(i, carry):
        rows_v[0, i // (DH // L), pl.ds((i % (DH // L)) * L, L)] = zero16
        return carry

    lax.fori_loop(0, CW * DH // L, zrows, 0)

    def zagg(j, carry):
        pltpu.sync_copy(rows_v.at[0],
                        agg_sh.at[pl.ds(s * ROWS_PER_TILE + j * CW, CW)])
        return carry

    lax.fori_loop(0, ROWS_PER_TILE // CW, zagg, 0)

    def zs(i, carry):
        s_v[pl.ds(i * L, L)] = zero16
        return carry

    lax.fori_loop(0, NPAD // L, zs, 0)
    pltpu.sync_copy(dinv_hbm, dinv_v)

    @pl.when(c == 0)
    def _():
        pltpu.sync_copy(src_hbm.at[pl.ds(s * K0, K0)], src_v)
        pltpu.sync_copy(dst_hbm.at[pl.ds(s * K0, K0)], dst_v)

    @pl.when(c == 1)
    def _():
        pltpu.sync_copy(src_hbm.at[pl.ds(NS * K0 + s * K1, K1)],
                        src_v.at[pl.ds(0, K1)])
        pltpu.sync_copy(dst_hbm.at[pl.ds(NS * K0 + s * K1, K1)],
                        dst_v.at[pl.ds(0, K1)])

    plsc.subcore_barrier()

    def gather(j, b):
        pltpu.async_copy(ht_hbm.at[src_v.at[j]], rows_v.at[b], gsem[b])

    def wait_gather(b):
        pltpu.make_async_copy(ht_hbm.at[src_v.at[0]], rows_v.at[b],
                              gsem[b]).wait()

    def scatter(j, b):
        pltpu.async_copy(rows_v.at[b], agg_sh.at[dst_v.at[j]], ssem[b],
                         add=True)

    def wait_scatter(b):
        pltpu.make_async_copy(rows_v.at[b], agg_sh.at[dst_v.at[0]],
                              ssem[b]).wait()

    def s_ops(j):
        def sv(i, c2):
            d16 = dst_v[j, pl.ds(i * L, L)]
            s16 = src_v[j, pl.ds(i * L, L)]
            vals = plsc.load_gather(dinv_v, [d16])
            plsc.addupdate_scatter(s_v, [s16], vals)
            return c2

        lax.fori_loop(0, CW // L, sv, 0)

    gather(0, 0)
    gather(1, 1)
    gather(2, 2)
    nq = k // 4

    def quad(g, carry):
        j0 = 4 * g
        for kk in range(4):
            j = j0 + kk
            wait_gather(kk)
            scatter(j, kk)
            s_ops(j)
            nb = (kk + 3) % 4
            if kk == 0:
                @pl.when(g >= 1)
                def _():
                    wait_scatter(nb)

                gather(j + 3, nb)
            else:
                @pl.when(g < nq - 1)
                def _():
                    wait_scatter(nb)
                    gather(j + 3, nb)
        return carry

    lax.fori_loop(0, nq, quad, 0)
    for b in range(4):
        wait_scatter(b)

    pltpu.sync_copy(s_v, spart_hbm.at[wid])
    plsc.subcore_barrier()

    def drain(j, carry):
        r0 = s * ROWS_PER_TILE + j * CW
        pltpu.sync_copy(agg_sh.at[pl.ds(r0, CW)], rows_v.at[0])
        pltpu.sync_copy(rows_v.at[0], sagg_hbm.at[c, pl.ds(r0, CW)])
        return carry

    lax.fori_loop(0, ROWS_PER_TILE // CW, drain, 0)


def _edge_stage(src3d, dst3d, ht, dinv_flat):
    mesh = plsc.VectorSubcoreMesh(
        core_axis_name="c", subcore_axis_name="s", num_cores=NC, num_subcores=NS)
    f = pl.kernel(
        _edge_body,
        out_type=(
            jax.ShapeDtypeStruct((NC, NPAD, DH), jnp.float32),
            jax.ShapeDtypeStruct((NW, NPAD), jnp.float32),
        ),
        mesh=mesh,
        scratch_types=[
            pltpu.VMEM((K0, CW), jnp.int32),
            pltpu.VMEM((K0, CW), jnp.int32),
            pltpu.VMEM((4, CW, DH), jnp.float32),
            pltpu.VMEM((NPAD,), jnp.float32),
            pltpu.VMEM((NPAD,), jnp.float32),
            pltpu.SemaphoreType.DMA,
            pltpu.SemaphoreType.DMA,
            pltpu.SemaphoreType.DMA,
            pltpu.SemaphoreType.DMA,
            pltpu.SemaphoreType.DMA,
            pltpu.SemaphoreType.DMA,
            pltpu.SemaphoreType.DMA,
            pltpu.SemaphoreType.DMA,
            pltpu.VMEM_SHARED((NPAD, DH), jnp.float32),
        ],
        compiler_params=pltpu.CompilerParams(
            needs_layout_passes=False, use_tc_tiling_on_sc=False),
    )
    return f(src3d, dst3d, ht, dinv_flat)


def _k4_body(sagg_ref, ht_ref, dinv_ref, spart_ref, b1_ref, w2_ref,
             fcw_ref, fcb_ref, b2_ref, out_ref, acc_ref):
    i = pl.program_id(0)

    @pl.when(i == 0)
    def _():
        acc_ref[...] = jnp.zeros_like(acc_ref)

    dinv = dinv_ref[0]
    a = sagg_ref[0] + sagg_ref[1] + ht_ref[...]
    out1 = jnp.maximum(_row_scale(dinv, a) + b1_ref[...], 0.0)
    ssum = jnp.sum(spart_ref[...], axis=0, keepdims=True)
    lane = lax.broadcasted_iota(jnp.int32, (1, BLK), 1) + i * BLK
    cvec = jnp.where(lane < N, dinv * (ssum + dinv), 0.0)
    acc_ref[...] += jnp.dot(cvec, out1, preferred_element_type=jnp.float32)

    @pl.when(i == NBLK - 1)
    def _():
        g = jnp.dot(acc_ref[...] / N, w2_ref[...],
                    preferred_element_type=jnp.float32) + b2_ref[...]
        val = jnp.dot(g, fcw_ref[...],
                      preferred_element_type=jnp.float32) + fcb_ref[...]
        out_ref[...] = jax.nn.sigmoid(val)


def _final_stage(sagg, ht, dinv2d, spart, b1, W2, fc_w, fc_b, b2):
    return pl.pallas_call(
        _k4_body,
        grid=(NBLK,),
        in_specs=[
            pl.BlockSpec((NC, BLK, DH), lambda i: (0, i, 0)),
            pl.BlockSpec((BLK, DH), lambda i: (i, 0)),
            pl.BlockSpec((1, 1, BLK), lambda i: (i, 0, 0)),
            pl.BlockSpec((NW, BLK), lambda i: (0, i)),
            pl.BlockSpec((1, DH), lambda i: (0, 0)),
            pl.BlockSpec((DH, DH), lambda i: (0, 0)),
            pl.BlockSpec((DH, 1), lambda i: (0, 0)),
            pl.BlockSpec((1, 1), lambda i: (0, 0)),
            pl.BlockSpec((1, DH), lambda i: (0, 0)),
        ],
        out_specs=pl.BlockSpec((1, 1), lambda i: (0, 0)),
        out_shape=jax.ShapeDtypeStruct((1, 1), jnp.float32),
        scratch_shapes=[pltpu.VMEM((1, DH), jnp.float32)],
        compiler_params=pltpu.CompilerParams(
            dimension_semantics=("arbitrary",)),
    )(sagg, ht, dinv2d, spart, b1, W2, fc_w, fc_b, b2)


def kernel(x, edge_index, W1, b1, W2, b2, fc_w, fc_b):
    src = edge_index[0].astype(jnp.int32)
    dst = edge_index[1].astype(jnp.int32)
    pad = jnp.full((EPAD - E,), N, dtype=jnp.int32)
    src_p = jnp.concatenate([src, pad])
    dst_p = jnp.concatenate([dst, pad])
    src3d = src_p.reshape(NCHUNKS, CW)
    dst3d = dst_p.reshape(NCHUNKS, CW)

    x_pad = jnp.pad(x, ((0, NPAD - N), (0, 0)))

    deg_part = _deg_counts(dst_p)
    ht, dinv2d = _scale_stage(x_pad, W1, deg_part)
    sagg, spart = _edge_stage(src3d, dst3d, ht, dinv2d.reshape(NPAD))
    out = _final_stage(sagg, ht, dinv2d, spart,
                       b1.reshape(1, DH), W2, fc_w, fc_b.reshape(1, 1),
                       b2.reshape(1, DH))
    return out.reshape(1)


# speedup vs baseline: 1.0344x; 1.0344x over previous
import functools

import jax
import jax.numpy as jnp
from jax import lax
from jax.experimental import pallas as pl
from jax.experimental.pallas import tpu as pltpu
from jax.experimental.pallas import tpu_sc as plsc

N = 10000
DIN = 128
DH = 64
E = 320000

NC = 2
NS = 16
NW = NC * NS
L = 16

NPAD = 10240
BLK = 1024
NBLK = NPAD // BLK
CW = 128
K0 = 108
K1 = 52
NCHUNKS = NS * (K0 + K1)
EPAD = NCHUNKS * CW
ROWS_PER_TILE = NPAD // NS

EW = EPAD // NW


def _deg_body(dst_hbm, out_hbm, idx_v, deg_v):
    c = lax.axis_index("c")
    s = lax.axis_index("s")
    wid = s * NC + c
    zero16 = jnp.zeros((L,), jnp.float32)

    def zb(i, carry):
        deg_v[pl.ds(i * L, L)] = zero16
        return carry

    lax.fori_loop(0, NPAD // L, zb, 0)
    pltpu.sync_copy(dst_hbm.at[pl.ds(wid * EW, EW)], idx_v)
    ones16 = jnp.ones((L,), jnp.float32)

    def body(i, carry):
        idx16 = idx_v[pl.ds(i * L, L)]
        plsc.addupdate_scatter(deg_v, [idx16], ones16)
        return carry

    lax.fori_loop(0, EW // L, body, 0)
    pltpu.sync_copy(deg_v, out_hbm.at[wid])


def _deg_counts(dst_flat):
    mesh = plsc.VectorSubcoreMesh(
        core_axis_name="c", subcore_axis_name="s", num_cores=NC, num_subcores=NS)
    f = pl.kernel(
        _deg_body,
        out_type=jax.ShapeDtypeStruct((NW, NPAD), jnp.float32),
        mesh=mesh,
        scratch_types=[
            pltpu.VMEM((EW,), jnp.int32),
            pltpu.VMEM((NPAD,), jnp.float32),
        ],
        compiler_params=pltpu.CompilerParams(
            needs_layout_passes=False, use_tc_tiling_on_sc=False),
    )
    return f(dst_flat)


def _row_scale(dinv_row, a):
    n = dinv_row.shape[1]
    outs = []
    rr = lax.broadcasted_iota(jnp.int32, (128, 128), 0)
    qq = lax.broadcasted_iota(jnp.int32, (128, 128), 1)
    eye = rr == qq
    for r in range(n // 128):
        dsub = dinv_row[:, r * 128:(r + 1) * 128]
        dg = jnp.where(eye, jnp.broadcast_to(dsub, (128, 128)), 0.0)
        outs.append(jnp.dot(dg, a[r * 128:(r + 1) * 128],
                            preferred_element_type=jnp.float32))
    return jnp.concatenate(outs, axis=0)


def _k2_body(x_ref, w1_ref, degp_ref, ht_ref, dinv_ref):
    deg = jnp.sum(degp_ref[...], axis=0, keepdims=True) + 1.0
    dinv = lax.rsqrt(deg)
    dinv_ref[...] = dinv.reshape(1, 1, BLK)
    h = jnp.dot(x_ref[...], w1_ref[...], preferred_element_type=jnp.float32)
    ht_ref[...] = _row_scale(dinv, h)


def _scale_stage(x_pad, W1, deg_part):
    return pl.pallas_call(
        _k2_body,
        grid=(NBLK,),
        in_specs=[
            pl.BlockSpec((BLK, DIN), lambda i: (i, 0)),
            pl.BlockSpec((DIN, DH), lambda i: (0, 0)),
            pl.BlockSpec((NW, BLK), lambda i: (0, i)),
        ],
        out_specs=[
            pl.BlockSpec((BLK, DH), lambda i: (i, 0)),
            pl.BlockSpec((1, 1, BLK), lambda i: (i, 0, 0)),
        ],
        out_shape=[
            jax.ShapeDtypeStruct((NPAD, DH), jnp.float32),
            jax.ShapeDtypeStruct((NBLK, 1, BLK), jnp.float32),
        ],
        compiler_params=pltpu.CompilerParams(
            dimension_semantics=("arbitrary",)),
    )(x_pad, W1, deg_part)


def _edge_body(src_hbm, dst_hbm, ht_hbm, dinv_hbm, sagg_hbm, spart_hbm,
               src_v, dst_v, rows_v, dinv_v, s_v,
               g0, g1, g2, g3, s0, s1, s2, s3, agg_sh):
    c = lax.axis_index("c")
    s = lax.axis_index("s")
    wid = s * NC + c
    gsem = [g0, g1, g2, g3]
    ssem = [s0, s1, s2, s3]
    zero16 = jnp.zeros((L,), jnp.float32)

    base = jnp.where(c == 0, s * K0, NS * K0 + s * K1)
    k = jnp.where(c == 0, K0, K1)

    def zrows(i, carry):
        rows_v[0, i // (DH // L), pl.ds((i % (DH // L)) * L, L)] = zero16
        return carry

    lax.fori_loop(0, CW * DH // L, zrows, 0)

    def zagg(j, carry):
        pltpu.sync_copy(rows_v.at[0],
                        agg_sh.at[pl.ds(s * ROWS_PER_TILE + j * CW, CW)])
        return carry

    lax.fori_loop(0, ROWS_PER_TILE // CW, zagg, 0)

    def zs(i, carry):
        s_v[pl.ds(i * L, L)] = zero16
        return carry

    lax.fori_loop(0, NPAD // L, zs, 0)
    pltpu.sync_copy(dinv_hbm, dinv_v)

    @pl.when(c == 0)
    def _():
        pltpu.sync_copy(src_hbm.at[pl.ds(s * K0, K0)], src_v)
        pltpu.sync_copy(dst_hbm.at[pl.ds(s * K0, K0)], dst_v)

    @pl.when(c == 1)
    def _():
        pltpu.sync_copy(src_hbm.at[pl.ds(NS * K0 + s * K1, K1)],
                        src_v.at[pl.ds(0, K1)])
        pltpu.sync_copy(dst_hbm.at[pl.ds(NS * K0 + s * K1, K1)],
                        dst_v.at[pl.ds(0, K1)])

    plsc.subcore_barrier()

    def gather(j, b):
        pltpu.async_copy(ht_hbm.at[src_v.at[j]], rows_v.at[b], gsem[b])

    def wait_gather(b):
        pltpu.make_async_copy(ht_hbm.at[src_v.at[0]], rows_v.at[b],
                              gsem[b]).wait()

    def scatter(j, b):
        pltpu.async_copy(rows_v.at[b], agg_sh.at[dst_v.at[j]], ssem[b],
                         add=True)

    def wait_scatter(b):
        pltpu.make_async_copy(rows_v.at[b], agg_sh.at[dst_v.at[0]],
                              ssem[b]).wait()

    def s_ops(j):
        def sv(i, c2):
            d16 = dst_v[j, pl.ds(i * L, L)]
            s16 = src_v[j, pl.ds(i * L, L)]
            vals = plsc.load_gather(dinv_v, [d16])
            plsc.addupdate_scatter(s_v, [s16], vals)
            return c2

        lax.fori_loop(0, CW // L, sv, 0)

    gather(0, 0)
    gather(1, 1)
    gather(2, 2)
    nq = k // 4

    def quad(g, carry):
        j0 = 4 * g
        for kk in range(4):
            j = j0 + kk
            wait_gather(kk)

            @pl.when(c == 0)
            def _():
                scatter(j, kk)

            @pl.when(c == 1)
            def _():
                pltpu.sync_copy(rows_v.at[kk], agg_sh.at[dst_v.at[j]],
                                add=True)

            s_ops(j)
            nb = (kk + 3) % 4
            if kk == 0:
                @pl.when((c == 0) & (g >= 1))
                def _():
                    wait_scatter(nb)

                gather(j + 3, nb)
            else:
                @pl.when(g < nq - 1)
                def _():
                    @pl.when(c == 0)
                    def _():
                        wait_scatter(nb)

                    gather(j + 3, nb)
        return carry

    lax.fori_loop(0, nq, quad, 0)

    @pl.when(c == 0)
    def _():
        for b in range(4):
            wait_scatter(b)

    pltpu.sync_copy(s_v, spart_hbm.at[wid])
    plsc.subcore_barrier()

    def drain(j, carry):
        r0 = s * ROWS_PER_TILE + j * CW
        pltpu.sync_copy(agg_sh.at[pl.ds(r0, CW)], rows_v.at[0])
        pltpu.sync_copy(rows_v.at[0], sagg_hbm.at[c, pl.ds(r0, CW)])
        return carry

    lax.fori_loop(0, ROWS_PER_TILE // CW, drain, 0)


def _edge_stage(src3d, dst3d, ht, dinv_flat):
    mesh = plsc.VectorSubcoreMesh(
        core_axis_name="c", subcore_axis_name="s", num_cores=NC, num_subcores=NS)
    f = pl.kernel(
        _edge_body,
        out_type=(
            jax.ShapeDtypeStruct((NC, NPAD, DH), jnp.float32),
            jax.ShapeDtypeStruct((NW, NPAD), jnp.float32),
        ),
        mesh=mesh,
        scratch_types=[
            pltpu.VMEM((K0, CW), jnp.int32),
            pltpu.VMEM((K0, CW), jnp.int32),
            pltpu.VMEM((4, CW, DH), jnp.float32),
            pltpu.VMEM((NPAD,), jnp.float32),
            pltpu.VMEM((NPAD,), jnp.float32),
            pltpu.SemaphoreType.DMA,
            pltpu.SemaphoreType.DMA,
            pltpu.SemaphoreType.DMA,
            pltpu.SemaphoreType.DMA,
            pltpu.SemaphoreType.DMA,
            pltpu.SemaphoreType.DMA,
            pltpu.SemaphoreType.DMA,
            pltpu.SemaphoreType.DMA,
            pltpu.VMEM_SHARED((NPAD, DH), jnp.float32),
        ],
        compiler_params=pltpu.CompilerParams(
            needs_layout_passes=False, use_tc_tiling_on_sc=False),
    )
    return f(src3d, dst3d, ht, dinv_flat)


def _k4_body(sagg_ref, ht_ref, dinv_ref, spart_ref, b1_ref, w2_ref,
             fcw_ref, fcb_ref, b2_ref, out_ref, acc_ref):
    i = pl.program_id(0)

    @pl.when(i == 0)
    def _():
        acc_ref[...] = jnp.zeros_like(acc_ref)

    dinv = dinv_ref[0]
    a = sagg_ref[0] + sagg_ref[1] + ht_ref[...]
    out1 = jnp.maximum(_row_scale(dinv, a) + b1_ref[...], 0.0)
    ssum = jnp.sum(spart_ref[...], axis=0, keepdims=True)
    lane = lax.broadcasted_iota(jnp.int32, (1, BLK), 1) + i * BLK
    cvec = jnp.where(lane < N, dinv * (ssum + dinv), 0.0)
    acc_ref[...] += jnp.dot(cvec, out1, preferred_element_type=jnp.float32)

    @pl.when(i == NBLK - 1)
    def _():
        g = jnp.dot(acc_ref[...] / N, w2_ref[...],
                    preferred_element_type=jnp.float32) + b2_ref[...]
        val = jnp.dot(g, fcw_ref[...],
                      preferred_element_type=jnp.float32) + fcb_ref[...]
        out_ref[...] = jax.nn.sigmoid(val)


def _final_stage(sagg, ht, dinv2d, spart, b1, W2, fc_w, fc_b, b2):
    return pl.pallas_call(
        _k4_body,
        grid=(NBLK,),
        in_specs=[
            pl.BlockSpec((NC, BLK, DH), lambda i: (0, i, 0)),
            pl.BlockSpec((BLK, DH), lambda i: (i, 0)),
            pl.BlockSpec((1, 1, BLK), lambda i: (i, 0, 0)),
            pl.BlockSpec((NW, BLK), lambda i: (0, i)),
            pl.BlockSpec((1, DH), lambda i: (0, 0)),
            pl.BlockSpec((DH, DH), lambda i: (0, 0)),
            pl.BlockSpec((DH, 1), lambda i: (0, 0)),
            pl.BlockSpec((1, 1), lambda i: (0, 0)),
            pl.BlockSpec((1, DH), lambda i: (0, 0)),
        ],
        out_specs=pl.BlockSpec((1, 1), lambda i: (0, 0)),
        out_shape=jax.ShapeDtypeStruct((1, 1), jnp.float32),
        scratch_shapes=[pltpu.VMEM((1, DH), jnp.float32)],
        compiler_params=pltpu.CompilerParams(
            dimension_semantics=("arbitrary",)),
    )(sagg, ht, dinv2d, spart, b1, W2, fc_w, fc_b, b2)


def kernel(x, edge_index, W1, b1, W2, b2, fc_w, fc_b):
    src = edge_index[0].astype(jnp.int32)
    dst = edge_index[1].astype(jnp.int32)
    pad = jnp.full((EPAD - E,), N, dtype=jnp.int32)
    src_p = jnp.concatenate([src, pad])
    dst_p = jnp.concatenate([dst, pad])
    src3d = src_p.reshape(NCHUNKS, CW)
    dst3d = dst_p.reshape(NCHUNKS, CW)

    x_pad = jnp.pad(x, ((0, NPAD - N), (0, 0)))

    deg_part = _deg_counts(dst_p)
    ht, dinv2d = _scale_stage(x_pad, W1, deg_part)
    sagg, spart = _edge_stage(src3d, dst3d, ht, dinv2d.reshape(NPAD))
    out = _final_stage(sagg, ht, dinv2d, spart,
                       b1.reshape(1, DH), W2, fc_w, fc_b.reshape(1, 1),
                       b2.reshape(1, DH))
    return out.reshape(1)


# speedup vs baseline: 1.0356x; 1.0011x over previous
import functools

import jax
import jax.numpy as jnp
from jax import lax
from jax.experimental import pallas as pl
from jax.experimental.pallas import tpu as pltpu
from jax.experimental.pallas import tpu_sc as plsc

N = 10000
DIN = 128
DH = 64
E = 320000

NC = 2
NS = 16
NW = NC * NS
L = 16

NPAD = 10240
BLK = 1024
NBLK = NPAD // BLK
CW = 128
K0 = 108
K1 = 52
NCHUNKS = NS * (K0 + K1)
EPAD = NCHUNKS * CW
ROWS_PER_TILE = NPAD // NS

EW = EPAD // NW


def _deg_body(dst_hbm, out_hbm, idx_v, deg_v):
    c = lax.axis_index("c")
    s = lax.axis_index("s")
    wid = s * NC + c
    zero16 = jnp.zeros((L,), jnp.float32)

    def zb(i, carry):
        deg_v[pl.ds(i * L, L)] = zero16
        return carry

    lax.fori_loop(0, NPAD // L, zb, 0)
    pltpu.sync_copy(dst_hbm.at[pl.ds(wid * EW, EW)], idx_v)
    ones16 = jnp.ones((L,), jnp.float32)

    def body(i, carry):
        idx16 = idx_v[pl.ds(i * L, L)]
        plsc.addupdate_scatter(deg_v, [idx16], ones16)
        return carry

    lax.fori_loop(0, EW // L, body, 0)
    pltpu.sync_copy(deg_v, out_hbm.at[wid])


def _deg_counts(dst_flat):
    mesh = plsc.VectorSubcoreMesh(
        core_axis_name="c", subcore_axis_name="s", num_cores=NC, num_subcores=NS)
    f = pl.kernel(
        _deg_body,
        out_type=jax.ShapeDtypeStruct((NW, NPAD), jnp.float32),
        mesh=mesh,
        scratch_types=[
            pltpu.VMEM((EW,), jnp.int32),
            pltpu.VMEM((NPAD,), jnp.float32),
        ],
        compiler_params=pltpu.CompilerParams(
            needs_layout_passes=False, use_tc_tiling_on_sc=False),
    )
    return f(dst_flat)


def _row_scale(dinv_row, a):
    n = dinv_row.shape[1]
    outs = []
    rr = lax.broadcasted_iota(jnp.int32, (128, 128), 0)
    qq = lax.broadcasted_iota(jnp.int32, (128, 128), 1)
    eye = rr == qq
    for r in range(n // 128):
        dsub = dinv_row[:, r * 128:(r + 1) * 128]
        dg = jnp.where(eye, jnp.broadcast_to(dsub, (128, 128)), 0.0)
        outs.append(jnp.dot(dg, a[r * 128:(r + 1) * 128],
                            preferred_element_type=jnp.float32))
    return jnp.concatenate(outs, axis=0)


def _k2_body(x_ref, w1_ref, degp_ref, ht_ref, dinv_ref):
    deg = jnp.sum(degp_ref[...], axis=0, keepdims=True) + 1.0
    dinv = lax.rsqrt(deg)
    dinv_ref[...] = dinv.reshape(1, 1, BLK)
    h = jnp.dot(x_ref[...], w1_ref[...], preferred_element_type=jnp.float32)
    ht_ref[...] = _row_scale(dinv, h)


def _scale_stage(x_pad, W1, deg_part):
    return pl.pallas_call(
        _k2_body,
        grid=(NBLK,),
        in_specs=[
            pl.BlockSpec((BLK, DIN), lambda i: (i, 0)),
            pl.BlockSpec((DIN, DH), lambda i: (0, 0)),
            pl.BlockSpec((NW, BLK), lambda i: (0, i)),
        ],
        out_specs=[
            pl.BlockSpec((BLK, DH), lambda i: (i, 0)),
            pl.BlockSpec((1, 1, BLK), lambda i: (i, 0, 0)),
        ],
        out_shape=[
            jax.ShapeDtypeStruct((NPAD, DH), jnp.float32),
            jax.ShapeDtypeStruct((NBLK, 1, BLK), jnp.float32),
        ],
        compiler_params=pltpu.CompilerParams(
            dimension_semantics=("arbitrary",)),
    )(x_pad, W1, deg_part)


def _edge_body(src_hbm, dst_hbm, ht_hbm, dinv_hbm, sagg_hbm, spart_hbm,
               src_v, dst_v, rows_v, dinv_v, s_v,
               g0, g1, g2, g3, s0, s1, s2, s3, agg_sh):
    c = lax.axis_index("c")
    s = lax.axis_index("s")
    wid = s * NC + c
    gsem = [g0, g1, g2, g3]
    ssem = [s0, s1, s2, s3]
    zero16 = jnp.zeros((L,), jnp.float32)

    def zrows(i, carry):
        rows_v[0, i // (DH // L), pl.ds((i % (DH // L)) * L, L)] = zero16
        return carry

    lax.fori_loop(0, CW * DH // L, zrows, 0)

    def zagg(j, carry):
        pltpu.sync_copy(rows_v.at[0],
                        agg_sh.at[pl.ds(s * ROWS_PER_TILE + j * CW, CW)])
        return carry

    lax.fori_loop(0, ROWS_PER_TILE // CW, zagg, 0)

    def zs(i, carry):
        s_v[pl.ds(i * L, L)] = zero16
        return carry

    lax.fori_loop(0, NPAD // L, zs, 0)
    pltpu.sync_copy(dinv_hbm, dinv_v)

    @pl.when(c == 0)
    def _():
        pltpu.sync_copy(src_hbm.at[pl.ds(s * K0, K0)], src_v)
        pltpu.sync_copy(dst_hbm.at[pl.ds(s * K0, K0)], dst_v)

    @pl.when(c == 1)
    def _():
        pltpu.sync_copy(src_hbm.at[pl.ds(NS * K0 + s * K1, K1)],
                        src_v.at[pl.ds(0, K1)])
        pltpu.sync_copy(dst_hbm.at[pl.ds(NS * K0 + s * K1, K1)],
                        dst_v.at[pl.ds(0, K1)])

    plsc.subcore_barrier()

    def gather(j, b):
        pltpu.async_copy(ht_hbm.at[src_v.at[j]], rows_v.at[b], gsem[b])

    def wait_gather(b):
        pltpu.make_async_copy(ht_hbm.at[src_v.at[0]], rows_v.at[b],
                              gsem[b]).wait()

    def scatter(j, b):
        pltpu.async_copy(rows_v.at[b], agg_sh.at[dst_v.at[j]], ssem[b],
                         add=True)

    def wait_scatter(b):
        pltpu.make_async_copy(rows_v.at[b], agg_sh.at[dst_v.at[0]],
                              ssem[b]).wait()

    def s_ops(j):
        def sv(i, c2):
            d16 = dst_v[j, pl.ds(i * L, L)]
            s16 = src_v[j, pl.ds(i * L, L)]
            vals = plsc.load_gather(dinv_v, [d16])
            plsc.addupdate_scatter(s_v, [s16], vals)
            return c2

        lax.fori_loop(0, CW // L, sv, 0)

    @pl.when(c == 0)
    def _():
        gather(0, 0)
        gather(1, 1)
        gather(2, 2)
        nq = K0 // 4

        def quad(g, carry):
            j0 = 4 * g
            for kk in range(4):
                j = j0 + kk
                wait_gather(kk)
                scatter(j, kk)
                s_ops(j)
                nb = (kk + 3) % 4
                if kk == 0:
                    @pl.when(g >= 1)
                    def _():
                        wait_scatter(nb)

                    gather(j + 3, nb)
                else:
                    @pl.when(g < nq - 1)
                    def _():
                        wait_scatter(nb)
                        gather(j + 3, nb)
            return carry

        lax.fori_loop(0, nq, quad, 0)
        for b in range(4):
            wait_scatter(b)

    @pl.when(c == 1)
    def _():
        gather(0, 0)

        def pair(g, carry):
            j0 = 2 * g
            gather(j0 + 1, 1)
            wait_gather(0)
            pltpu.sync_copy(rows_v.at[0], agg_sh.at[dst_v.at[j0]], add=True)
            s_ops(j0)

            @pl.when(j0 + 2 < K1)
            def _():
                gather(j0 + 2, 0)

            wait_gather(1)
            pltpu.sync_copy(rows_v.at[1], agg_sh.at[dst_v.at[j0 + 1]],
                            add=True)
            s_ops(j0 + 1)
            return carry

        lax.fori_loop(0, K1 // 2, pair, 0)

    pltpu.sync_copy(s_v, spart_hbm.at[wid])
    plsc.subcore_barrier()

    def drain(j, carry):
        r0 = s * ROWS_PER_TILE + j * CW
        pltpu.sync_copy(agg_sh.at[pl.ds(r0, CW)], rows_v.at[0])
        pltpu.sync_copy(rows_v.at[0], sagg_hbm.at[c, pl.ds(r0, CW)])
        return carry

    lax.fori_loop(0, ROWS_PER_TILE // CW, drain, 0)


def _edge_stage(src3d, dst3d, ht, dinv_flat):
    mesh = plsc.VectorSubcoreMesh(
        core_axis_name="c", subcore_axis_name="s", num_cores=NC, num_subcores=NS)
    f = pl.kernel(
        _edge_body,
        out_type=(
            jax.ShapeDtypeStruct((NC, NPAD, DH), jnp.float32),
            jax.ShapeDtypeStruct((NW, NPAD), jnp.float32),
        ),
        mesh=mesh,
        scratch_types=[
            pltpu.VMEM((K0, CW), jnp.int32),
            pltpu.VMEM((K0, CW), jnp.int32),
            pltpu.VMEM((4, CW, DH), jnp.float32),
            pltpu.VMEM((NPAD,), jnp.float32),
            pltpu.VMEM((NPAD,), jnp.float32),
            pltpu.SemaphoreType.DMA,
            pltpu.SemaphoreType.DMA,
            pltpu.SemaphoreType.DMA,
            pltpu.SemaphoreType.DMA,
            pltpu.SemaphoreType.DMA,
            pltpu.SemaphoreType.DMA,
            pltpu.SemaphoreType.DMA,
            pltpu.SemaphoreType.DMA,
            pltpu.VMEM_SHARED((NPAD, DH), jnp.float32),
        ],
        compiler_params=pltpu.CompilerParams(
            needs_layout_passes=False, use_tc_tiling_on_sc=False),
    )
    return f(src3d, dst3d, ht, dinv_flat)


def _k4_body(sagg_ref, ht_ref, dinv_ref, spart_ref, b1_ref, w2_ref,
             fcw_ref, fcb_ref, b2_ref, out_ref, acc_ref):
    i = pl.program_id(0)

    @pl.when(i == 0)
    def _():
        acc_ref[...] = jnp.zeros_like(acc_ref)

    dinv = dinv_ref[0]
    a = sagg_ref[0] + sagg_ref[1] + ht_ref[...]
    out1 = jnp.maximum(_row_scale(dinv, a) + b1_ref[...], 0.0)
    ssum = jnp.sum(spart_ref[...], axis=0, keepdims=True)
    lane = lax.broadcasted_iota(jnp.int32, (1, BLK), 1) + i * BLK
    cvec = jnp.where(lane < N, dinv * (ssum + dinv), 0.0)
    acc_ref[...] += jnp.dot(cvec, out1, preferred_element_type=jnp.float32)

    @pl.when(i == NBLK - 1)
    def _():
        g = jnp.dot(acc_ref[...] / N, w2_ref[...],
                    preferred_element_type=jnp.float32) + b2_ref[...]
        val = jnp.dot(g, fcw_ref[...],
                      preferred_element_type=jnp.float32) + fcb_ref[...]
        out_ref[...] = jax.nn.sigmoid(val)


def _final_stage(sagg, ht, dinv2d, spart, b1, W2, fc_w, fc_b, b2):
    return pl.pallas_call(
        _k4_body,
        grid=(NBLK,),
        in_specs=[
            pl.BlockSpec((NC, BLK, DH), lambda i: (0, i, 0)),
            pl.BlockSpec((BLK, DH), lambda i: (i, 0)),
            pl.BlockSpec((1, 1, BLK), lambda i: (i, 0, 0)),
            pl.BlockSpec((NW, BLK), lambda i: (0, i)),
            pl.BlockSpec((1, DH), lambda i: (0, 0)),
            pl.BlockSpec((DH, DH), lambda i: (0, 0)),
            pl.BlockSpec((DH, 1), lambda i: (0, 0)),
            pl.BlockSpec((1, 1), lambda i: (0, 0)),
            pl.BlockSpec((1, DH), lambda i: (0, 0)),
        ],
        out_specs=pl.BlockSpec((1, 1), lambda i: (0, 0)),
        out_shape=jax.ShapeDtypeStruct((1, 1), jnp.float32),
        scratch_shapes=[pltpu.VMEM((1, DH), jnp.float32)],
        compiler_params=pltpu.CompilerParams(
            dimension_semantics=("arbitrary",)),
    )(sagg, ht, dinv2d, spart, b1, W2, fc_w, fc_b, b2)


def kernel(x, edge_index, W1, b1, W2, b2, fc_w, fc_b):
    src = edge_index[0].astype(jnp.int32)
    dst = edge_index[1].astype(jnp.int32)
    pad = jnp.full((EPAD - E,), N, dtype=jnp.int32)
    src_p = jnp.concatenate([src, pad])
    dst_p = jnp.concatenate([dst, pad])
    src3d = src_p.reshape(NCHUNKS, CW)
    dst3d = dst_p.reshape(NCHUNKS, CW)

    x_pad = jnp.pad(x, ((0, NPAD - N), (0, 0)))

    deg_part = _deg_counts(dst_p)
    ht, dinv2d = _scale_stage(x_pad, W1, deg_part)
    sagg, spart = _edge_stage(src3d, dst3d, ht, dinv2d.reshape(NPAD))
    out = _final_stage(sagg, ht, dinv2d, spart,
                       b1.reshape(1, DH), W2, fc_w, fc_b.reshape(1, 1),
                       b2.reshape(1, DH))
    return out.reshape(1)


# speedup vs baseline: 1.6692x; 1.6118x over previous
import functools

import jax
import jax.numpy as jnp
from jax import lax
from jax.experimental import pallas as pl
from jax.experimental.pallas import tpu as pltpu
from jax.experimental.pallas import tpu_sc as plsc

N = 10000
DIN = 128
DH = 64
E = 320000

NC = 2
NS = 16
NW = NC * NS
L = 16

NPAD = 10240
BLK = 1024
NBLK = NPAD // BLK
CW = 128
NCHUNK = 79
EW = NCHUNK * CW
EPAD = NW * EW
ROWS_PER_TILE = NPAD // NS


def _deg_body(dst_hbm, out_hbm, idx_v, deg_v):
    c = lax.axis_index("c")
    s = lax.axis_index("s")
    wid = s * NC + c
    zero16 = jnp.zeros((L,), jnp.float32)

    def zb(i, carry):
        deg_v[pl.ds(i * L, L)] = zero16
        return carry

    lax.fori_loop(0, NPAD // L, zb, 0)
    pltpu.sync_copy(dst_hbm.at[pl.ds(wid * EW, EW)], idx_v)
    ones16 = jnp.ones((L,), jnp.float32)

    def body(i, carry):
        idx16 = idx_v[pl.ds(i * L, L)]
        plsc.addupdate_scatter(deg_v, [idx16], ones16)
        return carry

    lax.fori_loop(0, EW // L, body, 0)
    pltpu.sync_copy(deg_v, out_hbm.at[wid])


def _deg_counts(dst_flat):
    mesh = plsc.VectorSubcoreMesh(
        core_axis_name="c", subcore_axis_name="s", num_cores=NC, num_subcores=NS)
    f = pl.kernel(
        _deg_body,
        out_type=jax.ShapeDtypeStruct((NW, NPAD), jnp.float32),
        mesh=mesh,
        scratch_types=[
            pltpu.VMEM((EW,), jnp.int32),
            pltpu.VMEM((NPAD,), jnp.float32),
        ],
        compiler_params=pltpu.CompilerParams(
            needs_layout_passes=False, use_tc_tiling_on_sc=False),
    )
    return f(dst_flat)


def _k2_body(x_ref, w1_ref, degp_ref, ht_ref, dinv_ref):
    deg = jnp.sum(degp_ref[...], axis=0, keepdims=True) + 1.0
    dinv = lax.rsqrt(deg)
    dinv_ref[...] = dinv.reshape(1, 1, BLK)
    h = jnp.dot(x_ref[...], w1_ref[...], preferred_element_type=jnp.float32)
    r = lax.broadcasted_iota(jnp.int32, (BLK, BLK), 0)
    q = lax.broadcasted_iota(jnp.int32, (BLK, BLK), 1)
    diag = jnp.where(r == q, jnp.broadcast_to(dinv, (BLK, BLK)), 0.0)
    ht_ref[...] = jnp.dot(diag, h, preferred_element_type=jnp.float32)


def _scale_stage(x_pad, W1, deg_part):
    return pl.pallas_call(
        _k2_body,
        grid=(NBLK,),
        in_specs=[
            pl.BlockSpec((BLK, DIN), lambda i: (i, 0)),
            pl.BlockSpec((DIN, DH), lambda i: (0, 0)),
            pl.BlockSpec((NW, BLK), lambda i: (0, i)),
        ],
        out_specs=[
            pl.BlockSpec((BLK, DH), lambda i: (i, 0)),
            pl.BlockSpec((1, 1, BLK), lambda i: (i, 0, 0)),
        ],
        out_shape=[
            jax.ShapeDtypeStruct((NPAD, DH), jnp.float32),
            jax.ShapeDtypeStruct((NBLK, 1, BLK), jnp.float32),
        ],
        compiler_params=pltpu.CompilerParams(
            dimension_semantics=("arbitrary",)),
    )(x_pad, W1, deg_part)


def _edge_body(src_hbm, dst_hbm, ht_hbm, dinv_hbm, sagg_hbm, spart_hbm,
               src_v, dst_v, rows_v, dinv_v, s_v, sem0, sem1, agg_sh):
    c = lax.axis_index("c")
    s = lax.axis_index("s")
    wid = s * NC + c
    zero16 = jnp.zeros((L,), jnp.float32)

    def zrows(i, carry):
        rows_v[0, i // (DH // L), pl.ds((i % (DH // L)) * L, L)] = zero16
        return carry

    lax.fori_loop(0, CW * DH // L, zrows, 0)

    def zagg(j, carry):
        pltpu.sync_copy(rows_v.at[0],
                        agg_sh.at[pl.ds(s * ROWS_PER_TILE + j * CW, CW)])
        return carry

    lax.fori_loop(0, ROWS_PER_TILE // CW, zagg, 0)

    def zs(i, carry):
        s_v[pl.ds(i * L, L)] = zero16
        return carry

    lax.fori_loop(0, NPAD // L, zs, 0)
    pltpu.sync_copy(dinv_hbm, dinv_v)
    pltpu.sync_copy(src_hbm.at[wid], src_v)
    pltpu.sync_copy(dst_hbm.at[wid], dst_v)
    plsc.subcore_barrier()

    def gather(j, b, sem):
        pltpu.async_copy(ht_hbm.at[src_v.at[j]], rows_v.at[b], sem)

    def wait_gather(j, b, sem):
        pltpu.make_async_copy(ht_hbm.at[src_v.at[j]], rows_v.at[b], sem).wait()

    def process(j, b):
        pltpu.sync_copy(rows_v.at[b], agg_sh.at[dst_v.at[j]], add=True)

        def sv(k, c2):
            d16 = dst_v[j, pl.ds(k * L, L)]
            s16 = src_v[j, pl.ds(k * L, L)]
            vals = plsc.load_gather(dinv_v, [d16])
            plsc.addupdate_scatter(s_v, [s16], vals)
            return c2

        lax.fori_loop(0, CW // L, sv, 0)

    gather(0, 0, sem0)

    def pair(g, carry):
        j0 = 2 * g
        gather(j0 + 1, 1, sem1)
        wait_gather(j0, 0, sem0)
        process(j0, 0)
        gather(j0 + 2, 0, sem0)
        wait_gather(j0 + 1, 1, sem1)
        process(j0 + 1, 1)
        return carry

    lax.fori_loop(0, (NCHUNK - 1) // 2, pair, 0)
    wait_gather(NCHUNK - 1, 0, sem0)
    process(NCHUNK - 1, 0)

    pltpu.sync_copy(s_v, spart_hbm.at[wid])
    plsc.subcore_barrier()

    def drain(j, carry):
        r0 = s * ROWS_PER_TILE + j * CW
        pltpu.sync_copy(agg_sh.at[pl.ds(r0, CW)], rows_v.at[0])
        pltpu.sync_copy(rows_v.at[0], sagg_hbm.at[c, pl.ds(r0, CW)])
        return carry

    lax.fori_loop(0, ROWS_PER_TILE // CW, drain, 0)


def _edge_stage(src3d, dst3d, ht, dinv_flat):
    mesh = plsc.VectorSubcoreMesh(
        core_axis_name="c", subcore_axis_name="s", num_cores=NC, num_subcores=NS)
    f = pl.kernel(
        _edge_body,
        out_type=(
            jax.ShapeDtypeStruct((NC, NPAD, DH), jnp.float32),
            jax.ShapeDtypeStruct((NW, NPAD), jnp.float32),
        ),
        mesh=mesh,
        scratch_types=[
            pltpu.VMEM((NCHUNK, CW), jnp.int32),
            pltpu.VMEM((NCHUNK, CW), jnp.int32),
            pltpu.VMEM((2, CW, DH), jnp.float32),
            pltpu.VMEM((NPAD,), jnp.float32),
            pltpu.VMEM((NPAD,), jnp.float32),
            pltpu.SemaphoreType.DMA,
            pltpu.SemaphoreType.DMA,
            pltpu.VMEM_SHARED((NPAD, DH), jnp.float32),
        ],
        compiler_params=pltpu.CompilerParams(
            needs_layout_passes=False, use_tc_tiling_on_sc=False),
    )
    return f(src3d, dst3d, ht, dinv_flat)


def _k4_body(sagg_ref, ht_ref, dinv_ref, spart_ref, b1_ref, w2_ref,
             fcw_ref, fcb_ref, b2_ref, out_ref, acc_ref):
    i = pl.program_id(0)

    @pl.when(i == 0)
    def _():
        acc_ref[...] = jnp.zeros_like(acc_ref)

    dinv = dinv_ref[0]
    a = sagg_ref[0] + sagg_ref[1] + ht_ref[...]
    r = lax.broadcasted_iota(jnp.int32, (BLK, BLK), 0)
    q = lax.broadcasted_iota(jnp.int32, (BLK, BLK), 1)
    diag = jnp.where(r == q, jnp.broadcast_to(dinv, (BLK, BLK)), 0.0)
    out1 = jnp.maximum(
        jnp.dot(diag, a, preferred_element_type=jnp.float32) + b1_ref[...], 0.0)
    ssum = jnp.sum(spart_ref[...], axis=0, keepdims=True)
    lane = lax.broadcasted_iota(jnp.int32, (1, BLK), 1) + i * BLK
    cvec = jnp.where(lane < N, dinv * (ssum + dinv), 0.0)
    acc_ref[...] += jnp.dot(cvec, out1, preferred_element_type=jnp.float32)

    @pl.when(i == NBLK - 1)
    def _():
        g = jnp.dot(acc_ref[...] / N, w2_ref[...],
                    preferred_element_type=jnp.float32) + b2_ref[...]
        val = jnp.dot(g, fcw_ref[...],
                      preferred_element_type=jnp.float32) + fcb_ref[...]
        out_ref[...] = jax.nn.sigmoid(val)


def _final_stage(sagg, ht, dinv2d, spart, b1, W2, fc_w, fc_b, b2):
    return pl.pallas_call(
        _k4_body,
        grid=(NBLK,),
        in_specs=[
            pl.BlockSpec((NC, BLK, DH), lambda i: (0, i, 0)),
            pl.BlockSpec((BLK, DH), lambda i: (i, 0)),
            pl.BlockSpec((1, 1, BLK), lambda i: (i, 0, 0)),
            pl.BlockSpec((NW, BLK), lambda i: (0, i)),
            pl.BlockSpec((1, DH), lambda i: (0, 0)),
            pl.BlockSpec((DH, DH), lambda i: (0, 0)),
            pl.BlockSpec((DH, 1), lambda i: (0, 0)),
            pl.BlockSpec((1, 1), lambda i: (0, 0)),
            pl.BlockSpec((1, DH), lambda i: (0, 0)),
        ],
        out_specs=pl.BlockSpec((1, 1), lambda i: (0, 0)),
        out_shape=jax.ShapeDtypeStruct((1, 1), jnp.float32),
        scratch_shapes=[pltpu.VMEM((1, DH), jnp.float32)],
        compiler_params=pltpu.CompilerParams(
            dimension_semantics=("arbitrary",)),
    )(sagg, ht, dinv2d, spart, b1, W2, fc_w, fc_b, b2)


def kernel(x, edge_index, W1, b1, W2, b2, fc_w, fc_b):
    src = edge_index[0].astype(jnp.int32)
    dst = edge_index[1].astype(jnp.int32)
    pad = jnp.full((EPAD - E,), N, dtype=jnp.int32)
    src_p = jnp.concatenate([src, pad])
    dst_p = jnp.concatenate([dst, pad])
    src3d = src_p.reshape(NW, NCHUNK, CW)
    dst3d = dst_p.reshape(NW, NCHUNK, CW)

    x_pad = jnp.pad(x, ((0, NPAD - N), (0, 0)))

    deg_part = _deg_counts(dst_p)
    ht, dinv2d = _scale_stage(x_pad, W1, deg_part)
    sagg, spart = _edge_stage(src3d, dst3d, ht, dinv2d.reshape(NPAD))
    out = _final_stage(sagg, ht, dinv2d, spart,
                       b1.reshape(1, DH), W2, fc_w, fc_b.reshape(1, 1),
                       b2.reshape(1, DH))
    return out.reshape(1)


# speedup vs baseline: 1.6931x; 1.0143x over previous
import functools

import jax
import jax.numpy as jnp
from jax import lax
from jax.experimental import pallas as pl
from jax.experimental.pallas import tpu as pltpu
from jax.experimental.pallas import tpu_sc as plsc

N = 10000
DIN = 128
DH = 64
E = 320000

NC = 2
NS = 16
NW = NC * NS
L = 16

NPAD = 10240
BLK = 1024
NBLK = NPAD // BLK
CW = 128
NCHUNK = 79
EW = NCHUNK * CW
EPAD = NW * EW
ROWS_PER_TILE = NPAD // NS

STEAL = 19
KC0 = NCHUNK + STEAL
KC1 = NCHUNK - STEAL


def _deg_body(dst_hbm, out_hbm, idx_v, deg_v):
    c = lax.axis_index("c")
    s = lax.axis_index("s")
    wid = s * NC + c
    zero16 = jnp.zeros((L,), jnp.float32)

    def zb(i, carry):
        deg_v[pl.ds(i * L, L)] = zero16
        return carry

    lax.fori_loop(0, NPAD // L, zb, 0)
    pltpu.sync_copy(dst_hbm.at[pl.ds(wid * EW, EW)], idx_v)
    ones16 = jnp.ones((L,), jnp.float32)

    def body(i, carry):
        idx16 = idx_v[pl.ds(i * L, L)]
        plsc.addupdate_scatter(deg_v, [idx16], ones16)
        return carry

    lax.fori_loop(0, EW // L, body, 0)
    pltpu.sync_copy(deg_v, out_hbm.at[wid])


def _deg_counts(dst_flat):
    mesh = plsc.VectorSubcoreMesh(
        core_axis_name="c", subcore_axis_name="s", num_cores=NC, num_subcores=NS)
    f = pl.kernel(
        _deg_body,
        out_type=jax.ShapeDtypeStruct((NW, NPAD), jnp.float32),
        mesh=mesh,
        scratch_types=[
            pltpu.VMEM((EW,), jnp.int32),
            pltpu.VMEM((NPAD,), jnp.float32),
        ],
        compiler_params=pltpu.CompilerParams(
            needs_layout_passes=False, use_tc_tiling_on_sc=False),
    )
    return f(dst_flat)


def _k2_body(x_ref, w1_ref, degp_ref, ht_ref, dinv_ref):
    deg = jnp.sum(degp_ref[...], axis=0, keepdims=True) + 1.0
    dinv = lax.rsqrt(deg)
    dinv_ref[...] = dinv.reshape(1, 1, BLK)
    h = jnp.dot(x_ref[...], w1_ref[...], preferred_element_type=jnp.float32)
    r = lax.broadcasted_iota(jnp.int32, (BLK, BLK), 0)
    q = lax.broadcasted_iota(jnp.int32, (BLK, BLK), 1)
    diag = jnp.where(r == q, jnp.broadcast_to(dinv, (BLK, BLK)), 0.0)
    ht_ref[...] = jnp.dot(diag, h, preferred_element_type=jnp.float32)


def _scale_stage(x_pad, W1, deg_part):
    return pl.pallas_call(
        _k2_body,
        grid=(NBLK,),
        in_specs=[
            pl.BlockSpec((BLK, DIN), lambda i: (i, 0)),
            pl.BlockSpec((DIN, DH), lambda i: (0, 0)),
            pl.BlockSpec((NW, BLK), lambda i: (0, i)),
        ],
        out_specs=[
            pl.BlockSpec((BLK, DH), lambda i: (i, 0)),
            pl.BlockSpec((1, 1, BLK), lambda i: (i, 0, 0)),
        ],
        out_shape=[
            jax.ShapeDtypeStruct((NPAD, DH), jnp.float32),
            jax.ShapeDtypeStruct((NBLK, 1, BLK), jnp.float32),
        ],
        compiler_params=pltpu.CompilerParams(
            dimension_semantics=("arbitrary",)),
    )(x_pad, W1, deg_part)


def _edge_body(src_hbm, dst_hbm, ht_hbm, dinv_hbm, sagg_hbm, spart_hbm,
               src_v, dst_v, rows_v, dinv_v, s_v, sem0, sem1, agg_sh):
    c = lax.axis_index("c")
    s = lax.axis_index("s")
    wid = s * NC + c
    zero16 = jnp.zeros((L,), jnp.float32)

    def zrows(i, carry):
        rows_v[0, i // (DH // L), pl.ds((i % (DH // L)) * L, L)] = zero16
        return carry

    lax.fori_loop(0, CW * DH // L, zrows, 0)

    def zagg(j, carry):
        pltpu.sync_copy(rows_v.at[0],
                        agg_sh.at[pl.ds(s * ROWS_PER_TILE + j * CW, CW)])
        return carry

    lax.fori_loop(0, ROWS_PER_TILE // CW, zagg, 0)

    def zs(i, carry):
        s_v[pl.ds(i * L, L)] = zero16
        return carry

    lax.fori_loop(0, NPAD // L, zs, 0)
    pltpu.sync_copy(dinv_hbm, dinv_v)

    @pl.when(c == 0)
    def _():
        pltpu.sync_copy(src_hbm.at[pl.ds(wid * NCHUNK, NCHUNK)],
                        src_v.at[pl.ds(0, NCHUNK)])
        pltpu.sync_copy(dst_hbm.at[pl.ds(wid * NCHUNK, NCHUNK)],
                        dst_v.at[pl.ds(0, NCHUNK)])
        pltpu.sync_copy(src_hbm.at[pl.ds((wid + 1) * NCHUNK + KC1, STEAL)],
                        src_v.at[pl.ds(NCHUNK, STEAL)])
        pltpu.sync_copy(dst_hbm.at[pl.ds((wid + 1) * NCHUNK + KC1, STEAL)],
                        dst_v.at[pl.ds(NCHUNK, STEAL)])

    @pl.when(c == 1)
    def _():
        pltpu.sync_copy(src_hbm.at[pl.ds(wid * NCHUNK, KC1)],
                        src_v.at[pl.ds(0, KC1)])
        pltpu.sync_copy(dst_hbm.at[pl.ds(wid * NCHUNK, KC1)],
                        dst_v.at[pl.ds(0, KC1)])

    plsc.subcore_barrier()

    def gather(j, b, sem):
        pltpu.async_copy(ht_hbm.at[src_v.at[j]], rows_v.at[b], sem)

    def wait_gather(j, b, sem):
        pltpu.make_async_copy(ht_hbm.at[src_v.at[j]], rows_v.at[b], sem).wait()

    def process(j, b):
        pltpu.sync_copy(rows_v.at[b], agg_sh.at[dst_v.at[j]], add=True)

        def sv(k, c2):
            d16 = dst_v[j, pl.ds(k * L, L)]
            s16 = src_v[j, pl.ds(k * L, L)]
            vals = plsc.load_gather(dinv_v, [d16])
            plsc.addupdate_scatter(s_v, [s16], vals)
            return c2

        lax.fori_loop(0, CW // L, sv, 0)

    def run_chunks(kn):
        gather(0, 0, sem0)

        def pair(g, carry):
            j0 = 2 * g
            gather(j0 + 1, 1, sem1)
            wait_gather(j0, 0, sem0)
            process(j0, 0)

            @pl.when(j0 + 2 < kn)
            def _():
                gather(j0 + 2, 0, sem0)

            wait_gather(j0 + 1, 1, sem1)
            process(j0 + 1, 1)
            return carry

        lax.fori_loop(0, kn // 2, pair, 0)

    @pl.when(c == 0)
    def _():
        run_chunks(KC0)

    @pl.when(c == 1)
    def _():
        run_chunks(KC1)

    pltpu.sync_copy(s_v, spart_hbm.at[wid])
    plsc.subcore_barrier()

    def drain(j, carry):
        r0 = s * ROWS_PER_TILE + j * CW
        pltpu.sync_copy(agg_sh.at[pl.ds(r0, CW)], rows_v.at[0])
        pltpu.sync_copy(rows_v.at[0], sagg_hbm.at[c, pl.ds(r0, CW)])
        return carry

    lax.fori_loop(0, ROWS_PER_TILE // CW, drain, 0)


def _edge_stage(src3d, dst3d, ht, dinv_flat):
    mesh = plsc.VectorSubcoreMesh(
        core_axis_name="c", subcore_axis_name="s", num_cores=NC, num_subcores=NS)
    f = pl.kernel(
        _edge_body,
        out_type=(
            jax.ShapeDtypeStruct((NC, NPAD, DH), jnp.float32),
            jax.ShapeDtypeStruct((NW, NPAD), jnp.float32),
        ),
        mesh=mesh,
        scratch_types=[
            pltpu.VMEM((KC0, CW), jnp.int32),
            pltpu.VMEM((KC0, CW), jnp.int32),
            pltpu.VMEM((2, CW, DH), jnp.float32),
            pltpu.VMEM((NPAD,), jnp.float32),
            pltpu.VMEM((NPAD,), jnp.float32),
            pltpu.SemaphoreType.DMA,
            pltpu.SemaphoreType.DMA,
            pltpu.VMEM_SHARED((NPAD, DH), jnp.float32),
        ],
        compiler_params=pltpu.CompilerParams(
            needs_layout_passes=False, use_tc_tiling_on_sc=False),
    )
    return f(src3d, dst3d, ht, dinv_flat)


def _k4_body(sagg_ref, ht_ref, dinv_ref, spart_ref, b1_ref, w2_ref,
             fcw_ref, fcb_ref, b2_ref, out_ref, acc_ref):
    i = pl.program_id(0)

    @pl.when(i == 0)
    def _():
        acc_ref[...] = jnp.zeros_like(acc_ref)

    dinv = dinv_ref[0]
    a = sagg_ref[0] + sagg_ref[1] + ht_ref[...]
    r = lax.broadcasted_iota(jnp.int32, (BLK, BLK), 0)
    q = lax.broadcasted_iota(jnp.int32, (BLK, BLK), 1)
    diag = jnp.where(r == q, jnp.broadcast_to(dinv, (BLK, BLK)), 0.0)
    out1 = jnp.maximum(
        jnp.dot(diag, a, preferred_element_type=jnp.float32) + b1_ref[...], 0.0)
    ssum = jnp.sum(spart_ref[...], axis=0, keepdims=True)
    lane = lax.broadcasted_iota(jnp.int32, (1, BLK), 1) + i * BLK
    cvec = jnp.where(lane < N, dinv * (ssum + dinv), 0.0)
    acc_ref[...] += jnp.dot(cvec, out1, preferred_element_type=jnp.float32)

    @pl.when(i == NBLK - 1)
    def _():
        g = jnp.dot(acc_ref[...] / N, w2_ref[...],
                    preferred_element_type=jnp.float32) + b2_ref[...]
        val = jnp.dot(g, fcw_ref[...],
                      preferred_element_type=jnp.float32) + fcb_ref[...]
        out_ref[...] = jax.nn.sigmoid(val)


def _final_stage(sagg, ht, dinv2d, spart, b1, W2, fc_w, fc_b, b2):
    return pl.pallas_call(
        _k4_body,
        grid=(NBLK,),
        in_specs=[
            pl.BlockSpec((NC, BLK, DH), lambda i: (0, i, 0)),
            pl.BlockSpec((BLK, DH), lambda i: (i, 0)),
            pl.BlockSpec((1, 1, BLK), lambda i: (i, 0, 0)),
            pl.BlockSpec((NW, BLK), lambda i: (0, i)),
            pl.BlockSpec((1, DH), lambda i: (0, 0)),
            pl.BlockSpec((DH, DH), lambda i: (0, 0)),
            pl.BlockSpec((DH, 1), lambda i: (0, 0)),
            pl.BlockSpec((1, 1), lambda i: (0, 0)),
            pl.BlockSpec((1, DH), lambda i: (0, 0)),
        ],
        out_specs=pl.BlockSpec((1, 1), lambda i: (0, 0)),
        out_shape=jax.ShapeDtypeStruct((1, 1), jnp.float32),
        scratch_shapes=[pltpu.VMEM((1, DH), jnp.float32)],
        compiler_params=pltpu.CompilerParams(
            dimension_semantics=("arbitrary",)),
    )(sagg, ht, dinv2d, spart, b1, W2, fc_w, fc_b, b2)


def kernel(x, edge_index, W1, b1, W2, b2, fc_w, fc_b):
    src = edge_index[0].astype(jnp.int32)
    dst = edge_index[1].astype(jnp.int32)
    pad = jnp.full((EPAD - E,), N, dtype=jnp.int32)
    src_p = jnp.concatenate([src, pad])
    dst_p = jnp.concatenate([dst, pad])
    src3d = src_p.reshape(NW * NCHUNK, CW)
    dst3d = dst_p.reshape(NW * NCHUNK, CW)

    x_pad = jnp.pad(x, ((0, NPAD - N), (0, 0)))

    deg_part = _deg_counts(dst_p)
    ht, dinv2d = _scale_stage(x_pad, W1, deg_part)
    sagg, spart = _edge_stage(src3d, dst3d, ht, dinv2d.reshape(NPAD))
    out = _final_stage(sagg, ht, dinv2d, spart,
                       b1.reshape(1, DH), W2, fc_w, fc_b.reshape(1, 1),
                       b2.reshape(1, DH))
    return out.reshape(1)


# speedup vs baseline: 2.3258x; 1.3737x over previous
import functools

import jax
import jax.numpy as jnp
from jax import lax
from jax.experimental import pallas as pl
from jax.experimental.pallas import tpu as pltpu
from jax.experimental.pallas import tpu_sc as plsc

N = 10000
DIN = 128
DH = 64
E = 320000

NC = 2
NS = 16
NW = NC * NS
L = 16

NPAD = 10240
BLK = 1024
NBLK = NPAD // BLK
CW = 128
NCHUNK = 79
EW = NCHUNK * CW
EPAD = NW * EW
ROWS_PER_TILE = NPAD // NS

STEAL = 0
KC0 = NCHUNK + STEAL
KC1 = NCHUNK - STEAL


def _deg_body(dst_hbm, out_hbm, idx_v, deg_v):
    c = lax.axis_index("c")
    s = lax.axis_index("s")
    wid = s * NC + c
    zero16 = jnp.zeros((L,), jnp.float32)

    def zb(i, carry):
        deg_v[pl.ds(i * L, L)] = zero16
        return carry

    lax.fori_loop(0, NPAD // L, zb, 0)
    pltpu.sync_copy(dst_hbm.at[pl.ds(wid * EW, EW)], idx_v)
    ones16 = jnp.ones((L,), jnp.float32)

    def body(i, carry):
        idx16 = idx_v[pl.ds(i * L, L)]
        plsc.addupdate_scatter(deg_v, [idx16], ones16)
        return carry

    lax.fori_loop(0, EW // L, body, 0)
    pltpu.sync_copy(deg_v, out_hbm.at[wid])


def _deg_counts(dst_flat):
    mesh = plsc.VectorSubcoreMesh(
        core_axis_name="c", subcore_axis_name="s", num_cores=NC, num_subcores=NS)
    f = pl.kernel(
        _deg_body,
        out_type=jax.ShapeDtypeStruct((NW, NPAD), jnp.float32),
        mesh=mesh,
        scratch_types=[
            pltpu.VMEM((EW,), jnp.int32),
            pltpu.VMEM((NPAD,), jnp.float32),
        ],
        compiler_params=pltpu.CompilerParams(
            needs_layout_passes=False, use_tc_tiling_on_sc=False),
    )
    return f(dst_flat)


def _k2_body(x_ref, w1_ref, degp_ref, ht_ref, dinv_ref):
    deg = jnp.sum(degp_ref[...], axis=0, keepdims=True) + 1.0
    dinv = lax.rsqrt(deg)
    dinv_ref[...] = dinv.reshape(1, 1, BLK)
    h = jnp.dot(x_ref[...], w1_ref[...], preferred_element_type=jnp.float32)
    r = lax.broadcasted_iota(jnp.int32, (BLK, BLK), 0)
    q = lax.broadcasted_iota(jnp.int32, (BLK, BLK), 1)
    diag = jnp.where(r == q, jnp.broadcast_to(dinv, (BLK, BLK)), 0.0)
    ht_ref[...] = jnp.dot(diag, h, preferred_element_type=jnp.float32)


def _scale_stage(x_pad, W1, deg_part):
    return pl.pallas_call(
        _k2_body,
        grid=(NBLK,),
        in_specs=[
            pl.BlockSpec((BLK, DIN), lambda i: (i, 0)),
            pl.BlockSpec((DIN, DH), lambda i: (0, 0)),
            pl.BlockSpec((NW, BLK), lambda i: (0, i)),
        ],
        out_specs=[
            pl.BlockSpec((BLK, DH), lambda i: (i, 0)),
            pl.BlockSpec((1, 1, BLK), lambda i: (i, 0, 0)),
        ],
        out_shape=[
            jax.ShapeDtypeStruct((NPAD, DH), jnp.float32),
            jax.ShapeDtypeStruct((NBLK, 1, BLK), jnp.float32),
        ],
        compiler_params=pltpu.CompilerParams(
            dimension_semantics=("arbitrary",)),
    )(x_pad, W1, deg_part)


def _edge_body(src_hbm, dst_hbm, ht_hbm, dinv_hbm, sagg_hbm, spart_hbm,
               src_v, dst_v, rows_v, dinv_v, s_v, sem0, sem1, agg_sh):
    c = lax.axis_index("c")
    s = lax.axis_index("s")
    wid = s * NC + c
    zero16 = jnp.zeros((L,), jnp.float32)

    def zrows(i, carry):
        rows_v[0, i // (DH // L), pl.ds((i % (DH // L)) * L, L)] = zero16
        return carry

    lax.fori_loop(0, CW * DH // L, zrows, 0)

    def zagg(j, carry):
        pltpu.sync_copy(rows_v.at[0],
                        agg_sh.at[pl.ds(s * ROWS_PER_TILE + j * CW, CW)])
        return carry

    lax.fori_loop(0, ROWS_PER_TILE // CW, zagg, 0)

    def zs(i, carry):
        s_v[pl.ds(i * L, L)] = zero16
        return carry

    lax.fori_loop(0, NPAD // L, zs, 0)
    pltpu.sync_copy(dinv_hbm, dinv_v)

    if STEAL:
        @pl.when(c == 0)
        def _():
            pltpu.sync_copy(src_hbm.at[pl.ds(wid * NCHUNK, NCHUNK)],
                            src_v.at[pl.ds(0, NCHUNK)])
            pltpu.sync_copy(dst_hbm.at[pl.ds(wid * NCHUNK, NCHUNK)],
                            dst_v.at[pl.ds(0, NCHUNK)])
            pltpu.sync_copy(src_hbm.at[pl.ds((wid + 1) * NCHUNK + KC1, STEAL)],
                            src_v.at[pl.ds(NCHUNK, STEAL)])
            pltpu.sync_copy(dst_hbm.at[pl.ds((wid + 1) * NCHUNK + KC1, STEAL)],
                            dst_v.at[pl.ds(NCHUNK, STEAL)])

        @pl.when(c == 1)
        def _():
            pltpu.sync_copy(src_hbm.at[pl.ds(wid * NCHUNK, KC1)],
                            src_v.at[pl.ds(0, KC1)])
            pltpu.sync_copy(dst_hbm.at[pl.ds(wid * NCHUNK, KC1)],
                            dst_v.at[pl.ds(0, KC1)])
    else:
        pltpu.sync_copy(src_hbm.at[pl.ds(wid * NCHUNK, NCHUNK)], src_v)
        pltpu.sync_copy(dst_hbm.at[pl.ds(wid * NCHUNK, NCHUNK)], dst_v)

    plsc.subcore_barrier()

    def gather(j, b, sem):
        pltpu.async_copy(ht_hbm.at[src_v.at[j]], rows_v.at[b], sem)

    def wait_gather(j, b, sem):
        pltpu.make_async_copy(ht_hbm.at[src_v.at[j]], rows_v.at[b], sem).wait()

    def process(j, b):
        pltpu.sync_copy(rows_v.at[b], agg_sh.at[dst_v.at[j]], add=True)

        def sv(k, c2):
            d16 = dst_v[j, pl.ds(k * L, L)]
            s16 = src_v[j, pl.ds(k * L, L)]
            vals = plsc.load_gather(dinv_v, [d16])
            plsc.addupdate_scatter(s_v, [s16], vals)
            return c2

        lax.fori_loop(0, CW // L, sv, 0)

    def run_chunks(kn):
        gather(0, 0, sem0)

        def pair(g, carry):
            j0 = 2 * g
            gather(j0 + 1, 1, sem1)
            wait_gather(j0, 0, sem0)
            process(j0, 0)

            @pl.when(j0 + 2 < kn)
            def _():
                gather(j0 + 2, 0, sem0)

            wait_gather(j0 + 1, 1, sem1)
            process(j0 + 1, 1)
            return carry

        lax.fori_loop(0, kn // 2, pair, 0)
        if kn % 2:
            wait_gather(kn - 1, 0, sem0)
            process(kn - 1, 0)

    @pl.when(c == 0)
    def _():
        run_chunks(KC0)

    @pl.when(c == 1)
    def _():
        run_chunks(KC1)

    pltpu.sync_copy(s_v, spart_hbm.at[wid])
    plsc.subcore_barrier()

    def drain(j, carry):
        r0 = s * ROWS_PER_TILE + j * CW
        pltpu.sync_copy(agg_sh.at[pl.ds(r0, CW)], rows_v.at[0])
        pltpu.sync_copy(rows_v.at[0], sagg_hbm.at[c, pl.ds(r0, CW)])
        return carry

    lax.fori_loop(0, ROWS_PER_TILE // CW, drain, 0)


def _edge_stage(src3d, dst3d, ht, dinv_flat):
    mesh = plsc.VectorSubcoreMesh(
        core_axis_name="c", subcore_axis_name="s", num_cores=NC, num_subcores=NS)
    f = pl.kernel(
        _edge_body,
        out_type=(
            jax.ShapeDtypeStruct((NC, NPAD, DH), jnp.float32),
            jax.ShapeDtypeStruct((NW, NPAD), jnp.float32),
        ),
        mesh=mesh,
        scratch_types=[
            pltpu.VMEM((KC0, CW), jnp.int32),
            pltpu.VMEM((KC0, CW), jnp.int32),
            pltpu.VMEM((2, CW, DH), jnp.float32),
            pltpu.VMEM((NPAD,), jnp.float32),
            pltpu.VMEM((NPAD,), jnp.float32),
            pltpu.SemaphoreType.DMA,
            pltpu.SemaphoreType.DMA,
            pltpu.VMEM_SHARED((NPAD, DH), jnp.float32),
        ],
        compiler_params=pltpu.CompilerParams(
            needs_layout_passes=False, use_tc_tiling_on_sc=False),
    )
    return f(src3d, dst3d, ht, dinv_flat)


def _k4_body(sagg_ref, ht_ref, dinv_ref, spart_ref, b1_ref, w2_ref,
             fcw_ref, fcb_ref, b2_ref, out_ref, acc_ref):
    i = pl.program_id(0)

    @pl.when(i == 0)
    def _():
        acc_ref[...] = jnp.zeros_like(acc_ref)

    dinv = dinv_ref[0]
    a = sagg_ref[0] + sagg_ref[1] + ht_ref[...]
    r = lax.broadcasted_iota(jnp.int32, (BLK, BLK), 0)
    q = lax.broadcasted_iota(jnp.int32, (BLK, BLK), 1)
    diag = jnp.where(r == q, jnp.broadcast_to(dinv, (BLK, BLK)), 0.0)
    out1 = jnp.maximum(
        jnp.dot(diag, a, preferred_element_type=jnp.float32) + b1_ref[...], 0.0)
    ssum = jnp.sum(spart_ref[...], axis=0, keepdims=True)
    lane = lax.broadcasted_iota(jnp.int32, (1, BLK), 1) + i * BLK
    cvec = jnp.where(lane < N, dinv * (ssum + dinv), 0.0)
    acc_ref[...] += jnp.dot(cvec, out1, preferred_element_type=jnp.float32)

    @pl.when(i == NBLK - 1)
    def _():
        g = jnp.dot(acc_ref[...] / N, w2_ref[...],
                    preferred_element_type=jnp.float32) + b2_ref[...]
        val = jnp.dot(g, fcw_ref[...],
                      preferred_element_type=jnp.float32) + fcb_ref[...]
        out_ref[...] = jax.nn.sigmoid(val)


def _final_stage(sagg, ht, dinv2d, spart, b1, W2, fc_w, fc_b, b2):
    return pl.pallas_call(
        _k4_body,
        grid=(NBLK,),
        in_specs=[
            pl.BlockSpec((NC, BLK, DH), lambda i: (0, i, 0)),
            pl.BlockSpec((BLK, DH), lambda i: (i, 0)),
            pl.BlockSpec((1, 1, BLK), lambda i: (i, 0, 0)),
            pl.BlockSpec((NW, BLK), lambda i: (0, i)),
            pl.BlockSpec((1, DH), lambda i: (0, 0)),
            pl.BlockSpec((DH, DH), lambda i: (0, 0)),
            pl.BlockSpec((DH, 1), lambda i: (0, 0)),
            pl.BlockSpec((1, 1), lambda i: (0, 0)),
            pl.BlockSpec((1, DH), lambda i: (0, 0)),
        ],
        out_specs=pl.BlockSpec((1, 1), lambda i: (0, 0)),
        out_shape=jax.ShapeDtypeStruct((1, 1), jnp.float32),
        scratch_shapes=[pltpu.VMEM((1, DH), jnp.float32)],
        compiler_params=pltpu.CompilerParams(
            dimension_semantics=("arbitrary",)),
    )(sagg, ht, dinv2d, spart, b1, W2, fc_w, fc_b, b2)


def kernel(x, edge_index, W1, b1, W2, b2, fc_w, fc_b):
    src = edge_index[0].astype(jnp.int32)
    dst = edge_index[1].astype(jnp.int32)
    pad = N + (jnp.arange(EPAD - E, dtype=jnp.int32) % (NPAD - N))
    src_p = jnp.concatenate([src, pad])
    dst_p = jnp.concatenate([dst, pad])
    src3d = src_p.reshape(NW * NCHUNK, CW)
    dst3d = dst_p.reshape(NW * NCHUNK, CW)

    x_pad = jnp.pad(x, ((0, NPAD - N), (0, 0)))

    deg_part = _deg_counts(dst_p)
    ht, dinv2d = _scale_stage(x_pad, W1, deg_part)
    sagg, spart = _edge_stage(src3d, dst3d, ht, dinv2d.reshape(NPAD))
    out = _final_stage(sagg, ht, dinv2d, spart,
                       b1.reshape(1, DH), W2, fc_w, fc_b.reshape(1, 1),
                       b2.reshape(1, DH))
    return out.reshape(1)


# speedup vs baseline: 2.4120x; 1.0371x over previous
import functools

import jax
import jax.numpy as jnp
from jax import lax
from jax.experimental import pallas as pl
from jax.experimental.pallas import tpu as pltpu
from jax.experimental.pallas import tpu_sc as plsc

N = 10000
DIN = 128
DH = 64
E = 320000

NC = 2
NS = 16
NW = NC * NS
L = 16

NPAD = 10240
BLK = 1024
NBLK = NPAD // BLK
CW = 128
NCHUNK = 79
EW = NCHUNK * CW
EPAD = NW * EW
ROWS_PER_TILE = NPAD // NS

STEAL = 0
KC0 = NCHUNK + STEAL
KC1 = NCHUNK - STEAL


def _deg_body(dst_hbm, out_hbm, idx_v, deg_v):
    c = lax.axis_index("c")
    s = lax.axis_index("s")
    wid = s * NC + c
    zero16 = jnp.zeros((L,), jnp.float32)

    def zb(i, carry):
        deg_v[pl.ds(i * L, L)] = zero16
        return carry

    lax.fori_loop(0, NPAD // L, zb, 0)
    pltpu.sync_copy(dst_hbm.at[pl.ds(wid * EW, EW)], idx_v)
    ones16 = jnp.ones((L,), jnp.float32)

    def body(i, carry):
        idx16 = idx_v[pl.ds(i * L, L)]
        plsc.addupdate_scatter(deg_v, [idx16], ones16)
        return carry

    lax.fori_loop(0, EW // L, body, 0)
    pltpu.sync_copy(deg_v, out_hbm.at[wid])


def _deg_counts(dst_flat):
    mesh = plsc.VectorSubcoreMesh(
        core_axis_name="c", subcore_axis_name="s", num_cores=NC, num_subcores=NS)
    f = pl.kernel(
        _deg_body,
        out_type=jax.ShapeDtypeStruct((NW, NPAD), jnp.float32),
        mesh=mesh,
        scratch_types=[
            pltpu.VMEM((EW,), jnp.int32),
            pltpu.VMEM((NPAD,), jnp.float32),
        ],
        compiler_params=pltpu.CompilerParams(
            needs_layout_passes=False, use_tc_tiling_on_sc=False),
    )
    return f(dst_flat)


def _row_scale(dinv_row, a):
    n = dinv_row.shape[1]
    outs = []
    rr = lax.broadcasted_iota(jnp.int32, (128, 128), 0)
    qq = lax.broadcasted_iota(jnp.int32, (128, 128), 1)
    eye = rr == qq
    for r in range(n // 128):
        dsub = dinv_row[:, r * 128:(r + 1) * 128]
        dg = jnp.where(eye, jnp.broadcast_to(dsub, (128, 128)), 0.0)
        outs.append(jnp.dot(dg, a[r * 128:(r + 1) * 128],
                            preferred_element_type=jnp.float32))
    return jnp.concatenate(outs, axis=0)


def _k2_body(x_ref, w1_ref, degp_ref, ht_ref, dinv_ref):
    deg = jnp.sum(degp_ref[...], axis=0, keepdims=True) + 1.0
    dinv = lax.rsqrt(deg)
    dinv_ref[...] = dinv.reshape(1, 1, BLK)
    h = jnp.dot(x_ref[...], w1_ref[...], preferred_element_type=jnp.float32)
    ht_ref[...] = _row_scale(dinv, h)


def _scale_stage(x_pad, W1, deg_part):
    return pl.pallas_call(
        _k2_body,
        grid=(NBLK,),
        in_specs=[
            pl.BlockSpec((BLK, DIN), lambda i: (i, 0)),
            pl.BlockSpec((DIN, DH), lambda i: (0, 0)),
            pl.BlockSpec((NW, BLK), lambda i: (0, i)),
        ],
        out_specs=[
            pl.BlockSpec((BLK, DH), lambda i: (i, 0)),
            pl.BlockSpec((1, 1, BLK), lambda i: (i, 0, 0)),
        ],
        out_shape=[
            jax.ShapeDtypeStruct((NPAD, DH), jnp.float32),
            jax.ShapeDtypeStruct((NBLK, 1, BLK), jnp.float32),
        ],
        compiler_params=pltpu.CompilerParams(
            dimension_semantics=("arbitrary",)),
    )(x_pad, W1, deg_part)


def _edge_body(src_hbm, dst_hbm, ht_hbm, dinv_hbm, sagg_hbm, spart_hbm,
               src_v, dst_v, rows_v, dinv_v, s_v, sem0, sem1, agg_sh):
    c = lax.axis_index("c")
    s = lax.axis_index("s")
    wid = s * NC + c
    zero16 = jnp.zeros((L,), jnp.float32)

    def zrows(i, carry):
        rows_v[0, i // (DH // L), pl.ds((i % (DH // L)) * L, L)] = zero16
        return carry

    lax.fori_loop(0, CW * DH // L, zrows, 0)

    def zagg(j, carry):
        pltpu.sync_copy(rows_v.at[0],
                        agg_sh.at[pl.ds(s * ROWS_PER_TILE + j * CW, CW)])
        return carry

    lax.fori_loop(0, ROWS_PER_TILE // CW, zagg, 0)

    def zs(i, carry):
        s_v[pl.ds(i * L, L)] = zero16
        return carry

    lax.fori_loop(0, NPAD // L, zs, 0)
    pltpu.sync_copy(dinv_hbm, dinv_v)

    if STEAL:
        @pl.when(c == 0)
        def _():
            pltpu.sync_copy(src_hbm.at[pl.ds(wid * NCHUNK, NCHUNK)],
                            src_v.at[pl.ds(0, NCHUNK)])
            pltpu.sync_copy(dst_hbm.at[pl.ds(wid * NCHUNK, NCHUNK)],
                            dst_v.at[pl.ds(0, NCHUNK)])
            pltpu.sync_copy(src_hbm.at[pl.ds((wid + 1) * NCHUNK + KC1, STEAL)],
                            src_v.at[pl.ds(NCHUNK, STEAL)])
            pltpu.sync_copy(dst_hbm.at[pl.ds((wid + 1) * NCHUNK + KC1, STEAL)],
                            dst_v.at[pl.ds(NCHUNK, STEAL)])

        @pl.when(c == 1)
        def _():
            pltpu.sync_copy(src_hbm.at[pl.ds(wid * NCHUNK, KC1)],
                            src_v.at[pl.ds(0, KC1)])
            pltpu.sync_copy(dst_hbm.at[pl.ds(wid * NCHUNK, KC1)],
                            dst_v.at[pl.ds(0, KC1)])
    else:
        pltpu.sync_copy(src_hbm.at[pl.ds(wid * NCHUNK, NCHUNK)], src_v)
        pltpu.sync_copy(dst_hbm.at[pl.ds(wid * NCHUNK, NCHUNK)], dst_v)

    plsc.subcore_barrier()

    def gather(j, b, sem):
        pltpu.async_copy(ht_hbm.at[src_v.at[j]], rows_v.at[b], sem)

    def wait_gather(j, b, sem):
        pltpu.make_async_copy(ht_hbm.at[src_v.at[j]], rows_v.at[b], sem).wait()

    def process(j, b):
        pltpu.sync_copy(rows_v.at[b], agg_sh.at[dst_v.at[j]], add=True)

        def sv(k, c2):
            d16 = dst_v[j, pl.ds(k * L, L)]
            s16 = src_v[j, pl.ds(k * L, L)]
            vals = plsc.load_gather(dinv_v, [d16])
            plsc.addupdate_scatter(s_v, [s16], vals)
            return c2

        lax.fori_loop(0, CW // L, sv, 0)

    def run_chunks(kn):
        gather(0, 0, sem0)

        def pair(g, carry):
            j0 = 2 * g
            gather(j0 + 1, 1, sem1)
            wait_gather(j0, 0, sem0)
            process(j0, 0)

            @pl.when(j0 + 2 < kn)
            def _():
                gather(j0 + 2, 0, sem0)

            wait_gather(j0 + 1, 1, sem1)
            process(j0 + 1, 1)
            return carry

        lax.fori_loop(0, kn // 2, pair, 0)
        if kn % 2:
            wait_gather(kn - 1, 0, sem0)
            process(kn - 1, 0)

    @pl.when(c == 0)
    def _():
        run_chunks(KC0)

    @pl.when(c == 1)
    def _():
        run_chunks(KC1)

    pltpu.sync_copy(s_v, spart_hbm.at[wid])
    plsc.subcore_barrier()

    def drain(j, carry):
        r0 = s * ROWS_PER_TILE + j * CW
        pltpu.sync_copy(agg_sh.at[pl.ds(r0, CW)], rows_v.at[0])
        pltpu.sync_copy(rows_v.at[0], sagg_hbm.at[c, pl.ds(r0, CW)])
        return carry

    lax.fori_loop(0, ROWS_PER_TILE // CW, drain, 0)


def _edge_stage(src3d, dst3d, ht, dinv_flat):
    mesh = plsc.VectorSubcoreMesh(
        core_axis_name="c", subcore_axis_name="s", num_cores=NC, num_subcores=NS)
    f = pl.kernel(
        _edge_body,
        out_type=(
            jax.ShapeDtypeStruct((NC, NPAD, DH), jnp.float32),
            jax.ShapeDtypeStruct((NW, NPAD), jnp.float32),
        ),
        mesh=mesh,
        scratch_types=[
            pltpu.VMEM((KC0, CW), jnp.int32),
            pltpu.VMEM((KC0, CW), jnp.int32),
            pltpu.VMEM((2, CW, DH), jnp.float32),
            pltpu.VMEM((NPAD,), jnp.float32),
            pltpu.VMEM((NPAD,), jnp.float32),
            pltpu.SemaphoreType.DMA,
            pltpu.SemaphoreType.DMA,
            pltpu.VMEM_SHARED((NPAD, DH), jnp.float32),
        ],
        compiler_params=pltpu.CompilerParams(
            needs_layout_passes=False, use_tc_tiling_on_sc=False),
    )
    return f(src3d, dst3d, ht, dinv_flat)


def _k4_body(sagg_ref, ht_ref, dinv_ref, spart_ref, b1_ref, w2_ref,
             fcw_ref, fcb_ref, b2_ref, out_ref, acc_ref):
    i = pl.program_id(0)

    @pl.when(i == 0)
    def _():
        acc_ref[...] = jnp.zeros_like(acc_ref)

    dinv = dinv_ref[0]
    a = sagg_ref[0] + sagg_ref[1] + ht_ref[...]
    out1 = jnp.maximum(_row_scale(dinv, a) + b1_ref[...], 0.0)
    ssum = jnp.sum(spart_ref[...], axis=0, keepdims=True)
    lane = lax.broadcasted_iota(jnp.int32, (1, BLK), 1) + i * BLK
    cvec = jnp.where(lane < N, dinv * (ssum + dinv), 0.0)
    acc_ref[...] += jnp.dot(cvec, out1, preferred_element_type=jnp.float32)

    @pl.when(i == NBLK - 1)
    def _():
        g = jnp.dot(acc_ref[...] / N, w2_ref[...],
                    preferred_element_type=jnp.float32) + b2_ref[...]
        val = jnp.dot(g, fcw_ref[...],
                      preferred_element_type=jnp.float32) + fcb_ref[...]
        out_ref[...] = jax.nn.sigmoid(val)


def _final_stage(sagg, ht, dinv2d, spart, b1, W2, fc_w, fc_b, b2):
    return pl.pallas_call(
        _k4_body,
        grid=(NBLK,),
        in_specs=[
            pl.BlockSpec((NC, BLK, DH), lambda i: (0, i, 0)),
            pl.BlockSpec((BLK, DH), lambda i: (i, 0)),
            pl.BlockSpec((1, 1, BLK), lambda i: (i, 0, 0)),
            pl.BlockSpec((NW, BLK), lambda i: (0, i)),
            pl.BlockSpec((1, DH), lambda i: (0, 0)),
            pl.BlockSpec((DH, DH), lambda i: (0, 0)),
            pl.BlockSpec((DH, 1), lambda i: (0, 0)),
            pl.BlockSpec((1, 1), lambda i: (0, 0)),
            pl.BlockSpec((1, DH), lambda i: (0, 0)),
        ],
        out_specs=pl.BlockSpec((1, 1), lambda i: (0, 0)),
        out_shape=jax.ShapeDtypeStruct((1, 1), jnp.float32),
        scratch_shapes=[pltpu.VMEM((1, DH), jnp.float32)],
        compiler_params=pltpu.CompilerParams(
            dimension_semantics=("arbitrary",)),
    )(sagg, ht, dinv2d, spart, b1, W2, fc_w, fc_b, b2)


def kernel(x, edge_index, W1, b1, W2, b2, fc_w, fc_b):
    src = edge_index[0].astype(jnp.int32)
    dst = edge_index[1].astype(jnp.int32)
    pad = N + (jnp.arange(EPAD - E, dtype=jnp.int32) % (NPAD - N))
    src_p = jnp.concatenate([src, pad])
    dst_p = jnp.concatenate([dst, pad])
    src3d = src_p.reshape(NW * NCHUNK, CW)
    dst3d = dst_p.reshape(NW * NCHUNK, CW)

    x_pad = jnp.pad(x, ((0, NPAD - N), (0, 0)))

    deg_part = _deg_counts(dst_p)
    ht, dinv2d = _scale_stage(x_pad, W1, deg_part)
    sagg, spart = _edge_stage(src3d, dst3d, ht, dinv2d.reshape(NPAD))
    out = _final_stage(sagg, ht, dinv2d, spart,
                       b1.reshape(1, DH), W2, fc_w, fc_b.reshape(1, 1),
                       b2.reshape(1, DH))
    return out.reshape(1)


# speedup vs baseline: 2.6744x; 1.1088x over previous
import functools

import jax
import jax.numpy as jnp
from jax import lax
from jax.experimental import pallas as pl
from jax.experimental.pallas import tpu as pltpu
from jax.experimental.pallas import tpu_sc as plsc

N = 10000
DIN = 128
DH = 64
E = 320000

NC = 2
NS = 16
NW = NC * NS
L = 16

NPAD = 10240
BLK = 1024
NBLK = NPAD // BLK
CW = 128
NCHUNK = 80
EW = NCHUNK * CW
EPAD = NW * EW
ROWS_PER_TILE = NPAD // NS


def _deg_body(dst_hbm, out_hbm, idx_v, deg_v):
    c = lax.axis_index("c")
    s = lax.axis_index("s")
    wid = s * NC + c
    zero16 = jnp.zeros((L,), jnp.float32)

    def zb(i, carry):
        deg_v[pl.ds(i * L, L)] = zero16
        return carry

    lax.fori_loop(0, NPAD // L, zb, 0)
    pltpu.sync_copy(dst_hbm.at[pl.ds(wid * EW, EW)], idx_v)
    ones16 = jnp.ones((L,), jnp.float32)

    def body(i, carry):
        idx16 = idx_v[pl.ds(i * L, L)]
        plsc.addupdate_scatter(deg_v, [idx16], ones16)
        return carry

    lax.fori_loop(0, EW // L, body, 0)
    pltpu.sync_copy(deg_v, out_hbm.at[wid])


def _deg_counts(dst_flat):
    mesh = plsc.VectorSubcoreMesh(
        core_axis_name="c", subcore_axis_name="s", num_cores=NC, num_subcores=NS)
    f = pl.kernel(
        _deg_body,
        out_type=jax.ShapeDtypeStruct((NW, NPAD), jnp.float32),
        mesh=mesh,
        scratch_types=[
            pltpu.VMEM((EW,), jnp.int32),
            pltpu.VMEM((NPAD,), jnp.float32),
        ],
        compiler_params=pltpu.CompilerParams(
            needs_layout_passes=False, use_tc_tiling_on_sc=False),
    )
    return f(dst_flat)


def _row_scale(dinv_row, a):
    n = dinv_row.shape[1]
    outs = []
    rr = lax.broadcasted_iota(jnp.int32, (128, 128), 0)
    qq = lax.broadcasted_iota(jnp.int32, (128, 128), 1)
    eye = rr == qq
    for r in range(n // 128):
        dsub = dinv_row[:, r * 128:(r + 1) * 128]
        dg = jnp.where(eye, jnp.broadcast_to(dsub, (128, 128)), 0.0)
        outs.append(jnp.dot(dg, a[r * 128:(r + 1) * 128],
                            preferred_element_type=jnp.float32))
    return jnp.concatenate(outs, axis=0)


def _k2_body(x_ref, w1_ref, degp_ref, ht_ref, dinv_ref):
    deg = jnp.sum(degp_ref[...], axis=0, keepdims=True) + 1.0
    dinv = lax.rsqrt(deg)
    dinv_ref[...] = dinv.reshape(1, 1, BLK)
    h = jnp.dot(x_ref[...], w1_ref[...], preferred_element_type=jnp.float32)
    ht_ref[...] = _row_scale(dinv, h)


def _scale_stage(x_pad, W1, deg_part):
    return pl.pallas_call(
        _k2_body,
        grid=(NBLK,),
        in_specs=[
            pl.BlockSpec((BLK, DIN), lambda i: (i, 0)),
            pl.BlockSpec((DIN, DH), lambda i: (0, 0)),
            pl.BlockSpec((NW, BLK), lambda i: (0, i)),
        ],
        out_specs=[
            pl.BlockSpec((BLK, DH), lambda i: (i, 0)),
            pl.BlockSpec((1, 1, BLK), lambda i: (i, 0, 0)),
        ],
        out_shape=[
            jax.ShapeDtypeStruct((NPAD, DH), jnp.float32),
            jax.ShapeDtypeStruct((NBLK, 1, BLK), jnp.float32),
        ],
        compiler_params=pltpu.CompilerParams(
            dimension_semantics=("arbitrary",)),
    )(x_pad, W1, deg_part)


def _edge_body(src_hbm, dst_hbm, ht_hbm, dinv_hbm, sagg_hbm, spart_hbm,
               src_v, dst_v, rows_v, dinv_v, s_v,
               g0, g1, g2, g3, s0, s1, s2, s3, agg_sh):
    c = lax.axis_index("c")
    s = lax.axis_index("s")
    wid = s * NC + c
    gsem = [g0, g1, g2, g3]
    ssem = [s0, s1, s2, s3]
    zero16 = jnp.zeros((L,), jnp.float32)

    def zrows(i, carry):
        rows_v[0, i // (DH // L), pl.ds((i % (DH // L)) * L, L)] = zero16
        return carry

    lax.fori_loop(0, CW * DH // L, zrows, 0)

    def zagg(j, carry):
        pltpu.sync_copy(rows_v.at[0],
                        agg_sh.at[pl.ds(s * ROWS_PER_TILE + j * CW, CW)])
        return carry

    lax.fori_loop(0, ROWS_PER_TILE // CW, zagg, 0)

    def zs(i, carry):
        s_v[pl.ds(i * L, L)] = zero16
        return carry

    lax.fori_loop(0, NPAD // L, zs, 0)
    pltpu.sync_copy(dinv_hbm, dinv_v)
    pltpu.sync_copy(src_hbm.at[pl.ds(wid * NCHUNK, NCHUNK)], src_v)
    pltpu.sync_copy(dst_hbm.at[pl.ds(wid * NCHUNK, NCHUNK)], dst_v)
    plsc.subcore_barrier()

    def gather(j, b):
        pltpu.async_copy(ht_hbm.at[src_v.at[j]], rows_v.at[b], gsem[b])

    def wait_gather(b):
        pltpu.make_async_copy(ht_hbm.at[src_v.at[0]], rows_v.at[b],
                              gsem[b]).wait()

    def scatter(j, b):
        pltpu.async_copy(rows_v.at[b], agg_sh.at[dst_v.at[j]], ssem[b],
                         add=True)

    def wait_scatter(b):
        pltpu.make_async_copy(rows_v.at[b], agg_sh.at[dst_v.at[0]],
                              ssem[b]).wait()

    def s_ops(j):
        def sv(i, c2):
            d16 = dst_v[j, pl.ds(i * L, L)]
            s16 = src_v[j, pl.ds(i * L, L)]
            vals = plsc.load_gather(dinv_v, [d16])
            plsc.addupdate_scatter(s_v, [s16], vals)
            return c2

        lax.fori_loop(0, CW // L, sv, 0)

    gather(0, 0)
    gather(1, 1)
    gather(2, 2)
    nq = NCHUNK // 4

    def quad(g, carry):
        j0 = 4 * g
        for kk in range(4):
            j = j0 + kk
            wait_gather(kk)
            scatter(j, kk)
            s_ops(j)
            nb = (kk + 3) % 4
            if kk == 0:
                @pl.when(g >= 1)
                def _():
                    wait_scatter(nb)

                gather(j + 3, nb)
            else:
                @pl.when(g < nq - 1)
                def _():
                    wait_scatter(nb)
                    gather(j + 3, nb)
        return carry

    lax.fori_loop(0, nq, quad, 0)
    for b in range(4):
        wait_scatter(b)

    pltpu.sync_copy(s_v, spart_hbm.at[wid])
    plsc.subcore_barrier()

    def drain(j, carry):
        r0 = s * ROWS_PER_TILE + j * CW
        pltpu.sync_copy(agg_sh.at[pl.ds(r0, CW)], rows_v.at[0])
        pltpu.sync_copy(rows_v.at[0], sagg_hbm.at[c, pl.ds(r0, CW)])
        return carry

    lax.fori_loop(0, ROWS_PER_TILE // CW, drain, 0)


def _edge_stage(src3d, dst3d, ht, dinv_flat):
    mesh = plsc.VectorSubcoreMesh(
        core_axis_name="c", subcore_axis_name="s", num_cores=NC, num_subcores=NS)
    f = pl.kernel(
        _edge_body,
        out_type=(
            jax.ShapeDtypeStruct((NC, NPAD, DH), jnp.float32),
            jax.ShapeDtypeStruct((NW, NPAD), jnp.float32),
        ),
        mesh=mesh,
        scratch_types=[
            pltpu.VMEM((NCHUNK, CW), jnp.int32),
            pltpu.VMEM((NCHUNK, CW), jnp.int32),
            pltpu.VMEM((4, CW, DH), jnp.float32),
            pltpu.VMEM((NPAD,), jnp.float32),
            pltpu.VMEM((NPAD,), jnp.float32),
            pltpu.SemaphoreType.DMA,
            pltpu.SemaphoreType.DMA,
            pltpu.SemaphoreType.DMA,
            pltpu.SemaphoreType.DMA,
            pltpu.SemaphoreType.DMA,
            pltpu.SemaphoreType.DMA,
            pltpu.SemaphoreType.DMA,
            pltpu.SemaphoreType.DMA,
            pltpu.VMEM_SHARED((NPAD, DH), jnp.float32),
        ],
        compiler_params=pltpu.CompilerParams(
            needs_layout_passes=False, use_tc_tiling_on_sc=False),
    )
    return f(src3d, dst3d, ht, dinv_flat)


def _k4_body(sagg_ref, ht_ref, dinv_ref, spart_ref, b1_ref, w2_ref,
             fcw_ref, fcb_ref, b2_ref, out_ref, acc_ref):
    i = pl.program_id(0)

    @pl.when(i == 0)
    def _():
        acc_ref[...] = jnp.zeros_like(acc_ref)

    dinv = dinv_ref[0]
    a = sagg_ref[0] + sagg_ref[1] + ht_ref[...]
    out1 = jnp.maximum(_row_scale(dinv, a) + b1_ref[...], 0.0)
    ssum = jnp.sum(spart_ref[...], axis=0, keepdims=True)
    lane = lax.broadcasted_iota(jnp.int32, (1, BLK), 1) + i * BLK
    cvec = jnp.where(lane < N, dinv * (ssum + dinv), 0.0)
    acc_ref[...] += jnp.dot(cvec, out1, preferred_element_type=jnp.float32)

    @pl.when(i == NBLK - 1)
    def _():
        g = jnp.dot(acc_ref[...] / N, w2_ref[...],
                    preferred_element_type=jnp.float32) + b2_ref[...]
        val = jnp.dot(g, fcw_ref[...],
                      preferred_element_type=jnp.float32) + fcb_ref[...]
        out_ref[...] = jax.nn.sigmoid(val)


def _final_stage(sagg, ht, dinv2d, spart, b1, W2, fc_w, fc_b, b2):
    return pl.pallas_call(
        _k4_body,
        grid=(NBLK,),
        in_specs=[
            pl.BlockSpec((NC, BLK, DH), lambda i: (0, i, 0)),
            pl.BlockSpec((BLK, DH), lambda i: (i, 0)),
            pl.BlockSpec((1, 1, BLK), lambda i: (i, 0, 0)),
            pl.BlockSpec((NW, BLK), lambda i: (0, i)),
            pl.BlockSpec((1, DH), lambda i: (0, 0)),
            pl.BlockSpec((DH, DH), lambda i: (0, 0)),
            pl.BlockSpec((DH, 1), lambda i: (0, 0)),
            pl.BlockSpec((1, 1), lambda i: (0, 0)),
            pl.BlockSpec((1, DH), lambda i: (0, 0)),
        ],
        out_specs=pl.BlockSpec((1, 1), lambda i: (0, 0)),
        out_shape=jax.ShapeDtypeStruct((1, 1), jnp.float32),
        scratch_shapes=[pltpu.VMEM((1, DH), jnp.float32)],
        compiler_params=pltpu.CompilerParams(
            dimension_semantics=("arbitrary",)),
    )(sagg, ht, dinv2d, spart, b1, W2, fc_w, fc_b, b2)


def kernel(x, edge_index, W1, b1, W2, b2, fc_w, fc_b):
    src = edge_index[0].astype(jnp.int32)
    dst = edge_index[1].astype(jnp.int32)
    pad = N + (jnp.arange(EPAD - E, dtype=jnp.int32) % (NPAD - N))
    src_p = jnp.concatenate([src, pad])
    dst_p = jnp.concatenate([dst, pad])
    src3d = src_p.reshape(NW * NCHUNK, CW)
    dst3d = dst_p.reshape(NW * NCHUNK, CW)

    x_pad = jnp.pad(x, ((0, NPAD - N), (0, 0)))

    deg_part = _deg_counts(dst_p)
    ht, dinv2d = _scale_stage(x_pad, W1, deg_part)
    sagg, spart = _edge_stage(src3d, dst3d, ht, dinv2d.reshape(NPAD))
    out = _final_stage(sagg, ht, dinv2d, spart,
                       b1.reshape(1, DH), W2, fc_w, fc_b.reshape(1, 1),
                       b2.reshape(1, DH))
    return out.reshape(1)
